# baseline (speedup 1.0000x reference)
import jax
import jax.numpy as jnp
from jax import lax
from jax.experimental import pallas as pl
from jax.experimental.pallas import tpu as pltpu
from jax.experimental.pallas import tpu_sc as plsc

N = 10000
E = 320000
H = 128
G = 64
DEPTH = 3

NW = 32
EW = E // NW
CH = 80
NCH = EW // CH
NG = NCH // 5
BINR = 80
NPAD = 10240
NH2 = NPAD // 2
ACCR = NH2 + 8

BRE = 1280
BRN = 2000

_F32 = jnp.float32
_I32 = jnp.int32


def _mesh():
    return plsc.VectorSubcoreMesh(core_axis_name="c", subcore_axis_name="s")


_SC_PARAMS = pltpu.CompilerParams(needs_layout_passes=False)


def _sc_gather_multi(tables, idxa, idxb):
    nt = len(tables)

    def body(*refs):
        tabs = refs[:nt]
        ia, ib = refs[nt], refs[nt + 1]
        outs = refs[nt + 2:nt + 2 + 2 * nt]
        rest = refs[nt + 2 + 2 * nt:]
        iva, ivb = rest[0], rest[1]
        rest = rest[2:]
        bufs = [list(rest[0:5]), list(rest[5:10])]
        gsem = [list(rest[10:15]), list(rest[15:20])]
        osem = [list(rest[20:25]), list(rest[25:30])]
        wid = lax.axis_index("s") * 2 + lax.axis_index("c")
        ebase = wid * EW
        pltpu.sync_copy(ia.at[pl.ds(ebase, EW)], iva)
        pltpu.sync_copy(ib.at[pl.ds(ebase, EW)], ivb)

        def stream(tab, iv, out):
            def chunk(j):
                return out.at[pl.ds(ebase + j * CH, CH)]

            for b in range(5):
                pltpu.async_copy(tab.at[iv.at[pl.ds(b * CH, CH)]],
                                 bufs[0][b], gsem[0][b])

            def emit(bank, g):
                ob_ = 1 - bank
                for b in range(5):
                    @pl.when(g > 0)
                    def _():
                        pltpu.make_async_copy(
                            bufs[ob_][b], chunk(0), osem[ob_][b]).wait()

                    @pl.when(g < NG - 1)
                    def _():
                        pltpu.async_copy(
                            tab.at[iv.at[pl.ds((5 * (g + 1) + b) * CH, CH)]],
                            bufs[ob_][b], gsem[ob_][b])
                for b in range(5):
                    pltpu.make_async_copy(
                        tab.at[iv.at[pl.ds((5 * g + b) * CH, CH)]],
                        bufs[bank][b], gsem[bank][b]).wait()
                    pltpu.async_copy(bufs[bank][b], chunk(5 * g + b),
                                     osem[bank][b])

            def grp(g, c):
                @pl.when(g % 2 == 0)
                def _():
                    emit(0, g)

                @pl.when(g % 2 == 1)
                def _():
                    emit(1, g)
                return c

            lax.fori_loop(0, NG, grp, 0)
            bank = (NG - 1) % 2
            for b in range(5):
                pltpu.make_async_copy(bufs[bank][b], chunk(0),
                                      osem[bank][b]).wait()

        for t in range(nt):
            stream(tabs[t], iva, outs[2 * t])
            stream(tabs[t], ivb, outs[2 * t + 1])

    scratch = (
        [pltpu.VMEM((EW,), _I32)] * 2
        + [pltpu.VMEM((CH, H), _F32)] * 10
        + [pltpu.SemaphoreType.DMA] * 20
    )
    out_t = tuple(jax.ShapeDtypeStruct((E, H), _F32) for _ in range(2 * nt))
    return pl.kernel(body, out_type=out_t, mesh=_mesh(),
                     compiler_params=_SC_PARAMS,
                     scratch_types=scratch)(*tables, idxa, idxb)


def _bin_update(bins, tmp, d, vv, is_max):
    ri = lax.shift_right_logical(d, 7)
    ci = lax.bitwise_and(d, 127)
    iot = lax.iota(_I32, 16)
    ones = jnp.ones((16,), _I32)
    zeros = jnp.zeros((16,), _I32)

    def round_(rem):
        remb = rem != 0
        plsc.store_scatter(tmp, [ri, ci], iot, mask=remb)
        back = plsc.load_gather(tmp, [ri, ci])
        win = jnp.logical_and(remb, back == iot)
        cur = plsc.load_gather(bins, [ri, ci])
        upd = jnp.maximum(cur, vv) if is_max else cur + vv
        plsc.store_scatter(bins, [ri, ci], upd, mask=win)
        return jnp.where(jnp.logical_and(remb, jnp.logical_not(win)),
                         ones, zeros)

    rem0 = round_(ones)

    @pl.when(jnp.any(rem0 != 0))
    def _():
        def rounds(r, rm):
            return round_(rm)

        lax.fori_loop(0, 15, rounds, rem0)


def _sc_segmax3(a0, a1, a2, dst):
    def body(a0_h, a1_h, a2_h, d_h, out, av0, av1, av2, d_v, b0, b1, b2,
             tmp):
        wid = lax.axis_index("s") * 2 + lax.axis_index("c")
        avs = (av0, av1, av2)
        bins = (b0, b1, b2)
        for ah, av in zip((a0_h, a1_h, a2_h), avs):
            pltpu.sync_copy(ah.at[pl.ds(wid * EW, EW)], av)
        pltpu.sync_copy(d_h.at[pl.ds(wid * EW, EW)], d_v)

        neg = jnp.full((16,), -1e30, _F32)

        def zb(r, c):
            for bb in bins:
                for k in range(8):
                    bb[r, pl.ds(k * 16, 16)] = neg
            return c

        lax.fori_loop(0, BINR, zb, 0)

        def edge(j, c):
            d = d_v[pl.ds(j * 16, 16)]
            for av, bb in zip(avs, bins):
                v = av[pl.ds(j * 16, 16)]
                _bin_update(bb, tmp, d, v, True)
            return c

        lax.fori_loop(0, EW // 16, edge, 0)
        for t, bb in enumerate(bins):
            pltpu.sync_copy(bb, out.at[t, wid])

    scratch = [
        pltpu.VMEM((EW,), _F32),
        pltpu.VMEM((EW,), _F32),
        pltpu.VMEM((EW,), _F32),
        pltpu.VMEM((EW,), _I32),
        pltpu.VMEM((BINR, 128), _F32),
        pltpu.VMEM((BINR, 128), _F32),
        pltpu.VMEM((BINR, 128), _F32),
        pltpu.VMEM((BINR, 128), _I32),
    ]
    out_t = jax.ShapeDtypeStruct((3, NW, BINR, 128), _F32)
    return pl.kernel(body, out_type=out_t, mesh=_mesh(),
                     compiler_params=_SC_PARAMS,
                     scratch_types=scratch)(a0, a1, a2, dst)


def _sc_expw3(a0, a1, a2, dst, mx3):
    def body(a0_h, a1_h, a2_h, d_h, mx_h, w0_o, w1_o, w2_o, wp_out,
             av0, av1, av2, d_v, mxv, b0, b1, b2, tmp):
        wid = lax.axis_index("s") * 2 + lax.axis_index("c")
        avs = (av0, av1, av2)
        bins = (b0, b1, b2)
        for ah, av in zip((a0_h, a1_h, a2_h), avs):
            pltpu.sync_copy(ah.at[pl.ds(wid * EW, EW)], av)
        pltpu.sync_copy(d_h.at[pl.ds(wid * EW, EW)], d_v)

        zero = jnp.zeros((16,), _F32)

        def zb(r, c):
            for bb in bins:
                for k in range(8):
                    bb[r, pl.ds(k * 16, 16)] = zero
            return c

        lax.fori_loop(0, BINR, zb, 0)

        for t, (av, bb) in enumerate(zip(avs, bins)):
            pltpu.sync_copy(mx_h.at[t], mxv)

            def edge(j, c):
                d = d_v[pl.ds(j * 16, 16)]
                aa = av[pl.ds(j * 16, 16)]
                ri = lax.shift_right_logical(d, 7)
                ci = lax.bitwise_and(d, 127)
                m = plsc.load_gather(mxv, [ri, ci])
                w = jnp.exp(aa - m)
                av[pl.ds(j * 16, 16)] = w
                _bin_update(bb, tmp, d, w, False)
                return c

            lax.fori_loop(0, EW // 16, edge, 0)

        for t, (av, wo) in enumerate(zip(avs, (w0_o, w1_o, w2_o))):
            pltpu.sync_copy(av, wo.at[pl.ds(wid * EW, EW)])
        for t, bb in enumerate(bins):
            pltpu.sync_copy(bb, wp_out.at[t, wid])

    scratch = [
        pltpu.VMEM((EW,), _F32),
        pltpu.VMEM((EW,), _F32),
        pltpu.VMEM((EW,), _F32),
        pltpu.VMEM((EW,), _I32),
        pltpu.VMEM((BINR, 128), _F32),
        pltpu.VMEM((BINR, 128), _F32),
        pltpu.VMEM((BINR, 128), _F32),
        pltpu.VMEM((BINR, 128), _F32),
        pltpu.VMEM((BINR, 128), _I32),
    ]
    out_t = (jax.ShapeDtypeStruct((E,), _F32),
             jax.ShapeDtypeStruct((E,), _F32),
             jax.ShapeDtypeStruct((E,), _F32),
             jax.ShapeDtypeStruct((3, NW, BINR, 128), _F32))
    return pl.kernel(body, out_type=out_t, mesh=_mesh(),
                     compiler_params=_SC_PARAMS,
                     scratch_types=scratch)(a0, a1, a2, dst, mx3)


def _sc_scatter3(msgs, idx):
    def body(m0, m1, m2, i_h, o0, o1, o2, *rest):
        rbuf = list(rest[0:4])
        ibuf = list(rest[4:8])
        rsem = list(rest[8:12])
        isem = list(rest[12:16])
        acc = rest[16]
        cid = lax.axis_index("c")
        sid = lax.axis_index("s")
        wid = sid * 2 + cid
        ebase = wid * EW

        zero = jnp.zeros((16,), _F32)

        for m_h, out in ((m0, o0), (m1, o1), (m2, o2)):
            def zrow(r, c):
                for k in range(H // 16):
                    rbuf[0][r, pl.ds(k * 16, 16)] = zero
                return c

            lax.fori_loop(0, CH, zrow, 0)

            def zacc(q, c):
                pltpu.sync_copy(rbuf[0], acc.at[pl.ds(sid * 640 + q * CH,
                                                      CH)])
                return c

            lax.fori_loop(0, 8, zacc, 0)
            plsc.subcore_barrier()

            def chunk(j):
                return m_h.at[pl.ds(ebase + j * CH, CH)]

            def ichunk(j):
                return i_h.at[pl.ds(ebase + j * CH, CH)]

            def start(b, j):
                pltpu.async_copy(chunk(j), rbuf[b], rsem[b])
                pltpu.async_copy(ichunk(j), ibuf[b], isem[b])

            def do_chunk(b, j):
                pltpu.make_async_copy(chunk(j), rbuf[b], rsem[b]).wait()
                pltpu.make_async_copy(ichunk(j), ibuf[b], isem[b]).wait()
                pltpu.sync_copy(rbuf[b], acc.at[ibuf[b]], add=True)

                @pl.when(j + 4 < NCH)
                def _():
                    start(b, j + 4)

            for b in range(4):
                start(b, b)

            def grp(g, c):
                for b in range(4):
                    do_chunk(b, 4 * g + b)
                return c

            lax.fori_loop(0, (NCH - 1) // 4, grp, 0)
            do_chunk(0, NCH - 1)
            plsc.subcore_barrier()

            def dump(q, c):
                r0 = sid * 640 + q * CH
                pltpu.sync_copy(acc.at[pl.ds(r0, CH)], rbuf[0])
                pltpu.sync_copy(rbuf[0], out.at[cid, pl.ds(r0, CH)])
                return c

            lax.fori_loop(0, 8, dump, 0)
            plsc.subcore_barrier()

    scratch = (
        [pltpu.VMEM((CH, H), _F32)] * 4
        + [pltpu.VMEM((CH,), _I32)] * 4
        + [pltpu.SemaphoreType.DMA] * 8
        + [pltpu.VMEM_SHARED((NPAD, H), _F32)]
    )
    out_t = tuple(jax.ShapeDtypeStruct((2, NPAD, H), _F32) for _ in range(3))
    return pl.kernel(body, out_type=out_t, mesh=_mesh(), compiler_params=_SC_PARAMS,
                     scratch_types=scratch)(msgs[0], msgs[1], msgs[2], idx)


_TC_PARAMS = pltpu.CompilerParams(
    dimension_semantics=("arbitrary", "arbitrary"))


def _gelu(x):
    return 0.5 * x * (1.0 + lax.erf(x * 0.7071067811865476))


def _mlp(xin, w1, b1, w2, b2):
    y = _gelu(jnp.dot(xin, w1, preferred_element_type=_F32) + b1)
    return jnp.dot(y, w2, preferred_element_type=_F32) + b2


def _full_spec(shape):
    nd = len(shape)
    return pl.BlockSpec(shape, lambda p, i: (0,) * nd)


def _bn_phases(p, i, y, rows, stats, out_ref, post=None):
    @pl.when(jnp.logical_and(p == 0, i == 0))
    def _():
        stats[...] = jnp.zeros_like(stats)

    @pl.when(p == 0)
    def _():
        stats[0:1, :] += jnp.sum(y, axis=0, keepdims=True)
        stats[1:2, :] += jnp.sum(y * y, axis=0, keepdims=True)
        out_ref[...] = y

    @pl.when(p == 1)
    def _():
        m = stats[0:1, :] / rows
        v = stats[1:2, :] / rows - m * m
        yn = (y - m) * lax.rsqrt(v + 1e-5)
        out_ref[...] = yn
        if post is not None:
            post(yn)


def _tc_red(part, is_max):
    def body(p_ref, o_ref):
        x = p_ref[...]
        o_ref[...] = jnp.max(x, axis=0) if is_max else jnp.sum(x, axis=0)

    return pl.pallas_call(
        body,
        grid=(1,),
        in_specs=[pl.BlockSpec((NW, BINR, 128), lambda i: (0, 0, 0))],
        out_specs=pl.BlockSpec((BINR, 128), lambda i: (0, 0)),
        out_shape=jax.ShapeDtypeStruct((BINR, 128), _F32),
    )(part)


def _tc_enc(xp, w1, b1, w2, b2, rows, br):
    cin = xp.shape[1]
    nb = rows // br

    def body(x_ref, w1r, b1r, w2r, b2r, out_ref, stats):
        p = pl.program_id(0)
        i = pl.program_id(1)
        y = _mlp(x_ref[...], w1r[...], b1r[...], w2r[...], b2r[...])
        _bn_phases(p, i, y, float(rows), stats, out_ref)

    return pl.pallas_call(
        body,
        grid=(2, nb),
        in_specs=[
            pl.BlockSpec((br, cin), lambda p, i: (i, 0)),
            _full_spec((cin, H)),
            _full_spec((H,)),
            _full_spec((H, H)),
            _full_spec((H,)),
        ],
        out_specs=pl.BlockSpec((br, H), lambda p, i: (i, 0)),
        out_shape=jax.ShapeDtypeStruct((rows, H), _F32),
        scratch_shapes=[pltpu.VMEM((8, H), _F32)],
        compiler_params=_TC_PARAMS,
    )(xp, w1, b1, w2, b2)


def _tc_logits(gs, gd, eh):
    nb = E // BRE

    def body(s_ref, d_ref, e_ref, o_ref):
        o_ref[...] = jnp.sum((s_ref[...] + e_ref[...]) * d_ref[...],
                             axis=1, keepdims=True)

    spec = pl.BlockSpec((BRE, H), lambda i: (i, 0))
    return pl.pallas_call(
        body,
        grid=(nb,),
        in_specs=[spec, spec, spec],
        out_specs=pl.BlockSpec((BRE, 1), lambda i: (i, 0)),
        out_shape=jax.ShapeDtypeStruct((E, 1), _F32),
    )(gs, gd, eh)


def _tc_msgw(gs, w):
    nb = E // BRE

    def body(s_ref, w_ref, o_ref):
        o_ref[...] = s_ref[...] * w_ref[...]

    return pl.pallas_call(
        body,
        grid=(nb,),
        in_specs=[
            pl.BlockSpec((BRE, H), lambda i: (i, 0)),
            pl.BlockSpec((BRE, 1), lambda i: (i, 0)),
        ],
        out_specs=pl.BlockSpec((BRE, H), lambda i: (i, 0)),
        out_shape=jax.ShapeDtypeStruct((E, H), _F32),
    )(gs, w)


def _tc_node_update(rows_p, wsum, nh, eps, w1, b1, w2, b2, bid2, want_gh):
    nb = N // BRN

    def body(rp_ref, ws_ref, nh_ref, eps_ref, w1r, b1r, w2r, b2r, bid_ref,
             out_ref, nz_ref, *rest):
        p = pl.program_id(0)
        i = pl.program_id(1)
        rp = rp_ref[...]
        denom = ws_ref[...] + 1e-16
        nz = (rp[0] + rp[1]) / denom
        nz_ref[...] = nz
        xin = (1.0 + eps_ref[...]) * nh_ref[...] + nz
        y = _mlp(xin, w1r[...], b1r[...], w2r[...], b2r[...])
        if want_gh:
            gh_ref, stats, acc = rest

            @pl.when(jnp.logical_and(p == 1, i == 0))
            def _():
                acc[...] = jnp.zeros_like(acc)

            def post(yn):
                oh = (bid_ref[...] ==
                      lax.broadcasted_iota(_I32, (1, G), 1)).astype(_F32)
                acc[...] += lax.dot_general(
                    oh, yn, (((0,), (0,)), ((), ())),
                    preferred_element_type=_F32)
                gh_ref[...] = acc[...]
        else:
            (stats,) = rest
            post = None
        _bn_phases(p, i, y, float(N), stats, out_ref, post=post)

    out_shape = [jax.ShapeDtypeStruct((N, H), _F32),
                 jax.ShapeDtypeStruct((N, H), _F32)]
    out_specs = [pl.BlockSpec((BRN, H), lambda p, i: (i, 0)),
                 pl.BlockSpec((BRN, H), lambda p, i: (i, 0))]
    scratch = [pltpu.VMEM((8, H), _F32)]
    if want_gh:
        out_shape.append(jax.ShapeDtypeStruct((G, H), _F32))
        out_specs.append(pl.BlockSpec((G, H), lambda p, i: (0, 0)))
        scratch.append(pltpu.VMEM((G, H), _F32))

    return pl.pallas_call(
        body,
        grid=(2, nb),
        in_specs=[
            pl.BlockSpec((2, BRN, H), lambda p, i: (0, i, 0)),
            pl.BlockSpec((BRN, 1), lambda p, i: (i, 0)),
            pl.BlockSpec((BRN, H), lambda p, i: (i, 0)),
            _full_spec((H,)),
            _full_spec((H, H)),
            _full_spec((H,)),
            _full_spec((H, H)),
            _full_spec((H,)),
            pl.BlockSpec((BRN, 1), lambda p, i: (i, 0)),
        ],
        out_specs=out_specs,
        out_shape=out_shape,
        scratch_shapes=scratch,
        compiler_params=_TC_PARAMS,
    )(rows_p, wsum, nh, eps, w1, b1, w2, b2, bid2)


def _tc_edge_update(eh, zs, zd, eps, w1, b1, w2, b2):
    nb = E // BRE

    def body(eh_ref, zs_ref, zd_ref, eps_ref, w1r, b1r, w2r, b2r, out_ref,
             stats):
        p = pl.program_id(0)
        i = pl.program_id(1)
        xin = ((1.0 + eps_ref[...]) * eh_ref[...] + zs_ref[...] - zd_ref[...])
        y = _mlp(xin, w1r[...], b1r[...], w2r[...], b2r[...])
        _bn_phases(p, i, y, float(E), stats, out_ref)

    spec = pl.BlockSpec((BRE, H), lambda p, i: (i, 0))
    return pl.pallas_call(
        body,
        grid=(2, nb),
        in_specs=[spec, spec, spec, _full_spec((H,)), _full_spec((H, H)),
                  _full_spec((H,)), _full_spec((H, H)), _full_spec((H,))],
        out_specs=spec,
        out_shape=jax.ShapeDtypeStruct((E, H), _F32),
        scratch_shapes=[pltpu.VMEM((8, H), _F32)],
        compiler_params=_TC_PARAMS,
    )(eh, zs, zd, eps, w1, b1, w2, b2)


_TAGS = ("n", "e", "g")


def kernel(x, edge_attr, weights, edge_index, batch_ids):
    src = edge_index[0]
    dst = edge_index[1]
    bid2 = batch_ids.reshape(N, 1)
    xp = jnp.pad(x, ((0, 0), (0, 16 - x.shape[1])))
    ep = jnp.pad(edge_attr, ((0, 0), (0, 8 - edge_attr.shape[1])))

    def enc(arr, layers, rows, br, cin):
        w1 = jnp.pad(layers[0]["W"], ((0, cin - layers[0]["W"].shape[0]),
                                      (0, 0)))
        return _tc_enc(arr, w1, layers[0]["b"], layers[1]["W"],
                       layers[1]["b"], rows, br)

    nh = {}
    eh = {}
    for tag in _TAGS:
        nh[tag] = enc(xp, weights["nf_lin_for_" + tag], N, BRN, 16)
        eh[tag] = enc(ep, weights["ef_lin_for_" + tag], E, BRE, 8)

    final = {}
    for li in range(DEPTH):
        last = li == DEPTH - 1
        g6 = _sc_gather_multi([nh["n"], nh["e"], nh["g"]], src, dst)
        gs = {"n": g6[0], "e": g6[2], "g": g6[4]}
        gd = {"n": g6[1], "e": g6[3], "g": g6[5]}
        a = {t: _tc_logits(gs[t], gd[t], eh[t]).reshape(E) for t in _TAGS}
        mxp3 = _sc_segmax3(a["n"], a["e"], a["g"], dst)
        mx3 = jnp.stack([_tc_red(mxp3[t], True) for t in range(3)])
        w0, w1, w2, wp3 = _sc_expw3(a["n"], a["e"], a["g"], dst, mx3)
        ws = {"n": w0, "e": w1, "g": w2}
        wsum = {t: _tc_red(wp3[ti], False).reshape(NPAD, 1)
                for ti, t in enumerate(_TAGS)}
        msgs = [_tc_msgw(gs[t], ws[t].reshape(E, 1)) for t in _TAGS]
        rows_all = _sc_scatter3(msgs, dst)
        nz = {}
        for ti, tag in enumerate(_TAGS):
            p = weights["gnn_" + tag]
            nl = p["nf_lin"]
            want_gh = tag == "g" and last
            res = _tc_node_update(rows_all[ti], wsum[tag], nh[tag],
                                  p["nf_eps"], nl[0]["W"], nl[0]["b"],
                                  nl[1]["W"], nl[1]["b"], bid2, want_gh)
            nh[tag], nz[tag] = res[0], res[1]
            if want_gh:
                final["g"] = res[2]
        if last:
            z2 = _sc_gather_multi([nz["e"]], src, dst)
            zg = {"e": (z2[0], z2[1])}
            upd = ("e",)
        else:
            z6 = _sc_gather_multi([nz["n"], nz["e"], nz["g"]], src, dst)
            zg = {"n": (z6[0], z6[1]), "e": (z6[2], z6[3]),
                  "g": (z6[4], z6[5])}
            upd = _TAGS
        for tag in upd:
            p = weights["gnn_" + tag]
            el = p["ef_lin"]
            zs, zd = zg[tag]
            eh[tag] = _tc_edge_update(eh[tag], zs, zd, p["ef_eps"],
                                      el[0]["W"], el[0]["b"],
                                      el[1]["W"], el[1]["b"])
    final["n"] = nh["n"]
    final["e"] = eh["e"]
    return (final["n"], final["e"], final["g"])


# speedup vs baseline: 1.0773x; 1.0773x over previous
import jax
import jax.numpy as jnp
from jax import lax
from jax.experimental import pallas as pl
from jax.experimental.pallas import tpu as pltpu
from jax.experimental.pallas import tpu_sc as plsc

N = 10000
E = 320000
H = 128
G = 64
DEPTH = 3

NW = 32
EW = E // NW
CH = 80
NCH = EW // CH
NG = NCH // 5
BINR = 80
NPAD = 10240
NH2 = NPAD // 2
ACCR = NH2 + 8

BRE = 1280
BRN = 2000

_F32 = jnp.float32
_I32 = jnp.int32


def _mesh():
    return plsc.VectorSubcoreMesh(core_axis_name="c", subcore_axis_name="s")


_SC_PARAMS = pltpu.CompilerParams(needs_layout_passes=False)


def _sc_gather_multi(tables, idxa, idxb):
    nt = len(tables)

    def body(*refs):
        tabs = refs[:nt]
        ia, ib = refs[nt], refs[nt + 1]
        outs = refs[nt + 2:nt + 2 + 2 * nt]
        rest = refs[nt + 2 + 2 * nt:]
        iva, ivb = rest[0], rest[1]
        rest = rest[2:]
        bufs = [list(rest[0:5]), list(rest[5:10])]
        gsem = [list(rest[10:15]), list(rest[15:20])]
        osem = [list(rest[20:25]), list(rest[25:30])]
        wid = lax.axis_index("s") * 2 + lax.axis_index("c")
        ebase = wid * EW
        pltpu.sync_copy(ia.at[pl.ds(ebase, EW)], iva)
        pltpu.sync_copy(ib.at[pl.ds(ebase, EW)], ivb)

        def stream(tab, iv, out):
            def chunk(j):
                return out.at[pl.ds(ebase + j * CH, CH)]

            for b in range(5):
                pltpu.async_copy(tab.at[iv.at[pl.ds(b * CH, CH)]],
                                 bufs[0][b], gsem[0][b])

            def emit(bank, g):
                ob_ = 1 - bank
                for b in range(5):
                    @pl.when(g > 0)
                    def _():
                        pltpu.make_async_copy(
                            bufs[ob_][b], chunk(0), osem[ob_][b]).wait()

                    @pl.when(g < NG - 1)
                    def _():
                        pltpu.async_copy(
                            tab.at[iv.at[pl.ds((5 * (g + 1) + b) * CH, CH)]],
                            bufs[ob_][b], gsem[ob_][b])
                for b in range(5):
                    pltpu.make_async_copy(
                        tab.at[iv.at[pl.ds((5 * g + b) * CH, CH)]],
                        bufs[bank][b], gsem[bank][b]).wait()
                    pltpu.async_copy(bufs[bank][b], chunk(5 * g + b),
                                     osem[bank][b])

            def grp(g, c):
                @pl.when(g % 2 == 0)
                def _():
                    emit(0, g)

                @pl.when(g % 2 == 1)
                def _():
                    emit(1, g)
                return c

            lax.fori_loop(0, NG, grp, 0)
            bank = (NG - 1) % 2
            for b in range(5):
                pltpu.make_async_copy(bufs[bank][b], chunk(0),
                                      osem[bank][b]).wait()

        for t in range(nt):
            stream(tabs[t], iva, outs[2 * t])
            stream(tabs[t], ivb, outs[2 * t + 1])

    scratch = (
        [pltpu.VMEM((EW,), _I32)] * 2
        + [pltpu.VMEM((CH, H), _F32)] * 10
        + [pltpu.SemaphoreType.DMA] * 20
    )
    out_t = tuple(jax.ShapeDtypeStruct((E, H), _F32) for _ in range(2 * nt))
    return pl.kernel(body, out_type=out_t, mesh=_mesh(),
                     compiler_params=_SC_PARAMS,
                     scratch_types=scratch)(*tables, idxa, idxb)


def _bin_update(bins, tmp, d, vv, is_max):
    ri = lax.shift_right_logical(d, 7)
    ci = lax.bitwise_and(d, 127)
    iot = lax.iota(_I32, 16)
    ones = jnp.ones((16,), _I32)
    zeros = jnp.zeros((16,), _I32)

    def round_(rem):
        remb = rem != 0
        plsc.store_scatter(tmp, [ri, ci], iot, mask=remb)
        back = plsc.load_gather(tmp, [ri, ci])
        win = jnp.logical_and(remb, back == iot)
        cur = plsc.load_gather(bins, [ri, ci])
        upd = jnp.maximum(cur, vv) if is_max else cur + vv
        plsc.store_scatter(bins, [ri, ci], upd, mask=win)
        return jnp.where(jnp.logical_and(remb, jnp.logical_not(win)),
                         ones, zeros)

    rem0 = round_(ones)

    @pl.when(jnp.any(rem0 != 0))
    def _():
        def rounds(r, rm):
            return round_(rm)

        lax.fori_loop(0, 15, rounds, rem0)


def _sc_segmax3(a0, a1, a2, dst):
    def body(a0_h, a1_h, a2_h, d_h, out, av0, av1, av2, d_v, b0, b1, b2,
             tmp):
        wid = lax.axis_index("s") * 2 + lax.axis_index("c")
        avs = (av0, av1, av2)
        bins = (b0, b1, b2)
        for ah, av in zip((a0_h, a1_h, a2_h), avs):
            pltpu.sync_copy(ah.at[pl.ds(wid * EW, EW)], av)
        pltpu.sync_copy(d_h.at[pl.ds(wid * EW, EW)], d_v)

        neg = jnp.full((16,), -1e30, _F32)

        def zb(r, c):
            for bb in bins:
                for k in range(8):
                    bb[r, pl.ds(k * 16, 16)] = neg
            return c

        lax.fori_loop(0, BINR, zb, 0)

        def edge(j, c):
            d = d_v[pl.ds(j * 16, 16)]
            for av, bb in zip(avs, bins):
                v = av[pl.ds(j * 16, 16)]
                _bin_update(bb, tmp, d, v, True)
            return c

        lax.fori_loop(0, EW // 16, edge, 0)
        for t, bb in enumerate(bins):
            pltpu.sync_copy(bb, out.at[t, wid])

    scratch = [
        pltpu.VMEM((EW,), _F32),
        pltpu.VMEM((EW,), _F32),
        pltpu.VMEM((EW,), _F32),
        pltpu.VMEM((EW,), _I32),
        pltpu.VMEM((BINR, 128), _F32),
        pltpu.VMEM((BINR, 128), _F32),
        pltpu.VMEM((BINR, 128), _F32),
        pltpu.VMEM((BINR, 128), _I32),
    ]
    out_t = jax.ShapeDtypeStruct((3, NW, BINR, 128), _F32)
    return pl.kernel(body, out_type=out_t, mesh=_mesh(),
                     compiler_params=_SC_PARAMS,
                     scratch_types=scratch)(a0, a1, a2, dst)


def _sc_expw3(a0, a1, a2, dst, mx3):
    def body(a0_h, a1_h, a2_h, d_h, mx_h, w0_o, w1_o, w2_o, wp_out,
             av0, av1, av2, d_v, mxv, b0, b1, b2, tmp):
        wid = lax.axis_index("s") * 2 + lax.axis_index("c")
        avs = (av0, av1, av2)
        bins = (b0, b1, b2)
        for ah, av in zip((a0_h, a1_h, a2_h), avs):
            pltpu.sync_copy(ah.at[pl.ds(wid * EW, EW)], av)
        pltpu.sync_copy(d_h.at[pl.ds(wid * EW, EW)], d_v)

        zero = jnp.zeros((16,), _F32)

        def zb(r, c):
            for bb in bins:
                for k in range(8):
                    bb[r, pl.ds(k * 16, 16)] = zero
            return c

        lax.fori_loop(0, BINR, zb, 0)

        for t, (av, bb) in enumerate(zip(avs, bins)):
            pltpu.sync_copy(mx_h.at[t], mxv)

            def edge(j, c):
                d = d_v[pl.ds(j * 16, 16)]
                aa = av[pl.ds(j * 16, 16)]
                ri = lax.shift_right_logical(d, 7)
                ci = lax.bitwise_and(d, 127)
                m = plsc.load_gather(mxv, [ri, ci])
                w = jnp.exp(aa - m)
                av[pl.ds(j * 16, 16)] = w
                _bin_update(bb, tmp, d, w, False)
                return c

            lax.fori_loop(0, EW // 16, edge, 0)

        for t, (av, wo) in enumerate(zip(avs, (w0_o, w1_o, w2_o))):
            pltpu.sync_copy(av, wo.at[pl.ds(wid * EW, EW)])
        for t, bb in enumerate(bins):
            pltpu.sync_copy(bb, wp_out.at[t, wid])

    scratch = [
        pltpu.VMEM((EW,), _F32),
        pltpu.VMEM((EW,), _F32),
        pltpu.VMEM((EW,), _F32),
        pltpu.VMEM((EW,), _I32),
        pltpu.VMEM((BINR, 128), _F32),
        pltpu.VMEM((BINR, 128), _F32),
        pltpu.VMEM((BINR, 128), _F32),
        pltpu.VMEM((BINR, 128), _F32),
        pltpu.VMEM((BINR, 128), _I32),
    ]
    out_t = (jax.ShapeDtypeStruct((E,), _F32),
             jax.ShapeDtypeStruct((E,), _F32),
             jax.ShapeDtypeStruct((E,), _F32),
             jax.ShapeDtypeStruct((3, NW, BINR, 128), _F32))
    return pl.kernel(body, out_type=out_t, mesh=_mesh(),
                     compiler_params=_SC_PARAMS,
                     scratch_types=scratch)(a0, a1, a2, dst, mx3)


def _sc_segmax1(a, dst):
    def body(a_h, d_h, out, a_v, d_v, bins, tmp):
        wid = lax.axis_index("s") * 2 + lax.axis_index("c")
        pltpu.sync_copy(a_h.at[pl.ds(wid * EW, EW)], a_v)
        pltpu.sync_copy(d_h.at[pl.ds(wid * EW, EW)], d_v)

        neg = jnp.full((16,), -1e30, _F32)

        def zb(r, c):
            for k in range(8):
                bins[r, pl.ds(k * 16, 16)] = neg
            return c

        lax.fori_loop(0, BINR, zb, 0)

        def edge(j, c):
            d = d_v[pl.ds(j * 16, 16)]
            v = a_v[pl.ds(j * 16, 16)]
            _bin_update(bins, tmp, d, v, True)
            return c

        lax.fori_loop(0, EW // 16, edge, 0)
        pltpu.sync_copy(bins, out.at[wid])

    scratch = [
        pltpu.VMEM((EW,), _F32),
        pltpu.VMEM((EW,), _I32),
        pltpu.VMEM((BINR, 128), _F32),
        pltpu.VMEM((BINR, 128), _I32),
    ]
    out_t = jax.ShapeDtypeStruct((NW, BINR, 128), _F32)
    return pl.kernel(body, out_type=out_t, mesh=_mesh(),
                     compiler_params=_SC_PARAMS,
                     scratch_types=scratch)(a, dst)


def _sc_expw1(a, dst, mx):
    def body(a_h, d_h, mx_h, w_o, wp_out, a_v, d_v, mxv, bins, tmp):
        wid = lax.axis_index("s") * 2 + lax.axis_index("c")
        pltpu.sync_copy(a_h.at[pl.ds(wid * EW, EW)], a_v)
        pltpu.sync_copy(d_h.at[pl.ds(wid * EW, EW)], d_v)
        pltpu.sync_copy(mx_h, mxv)

        zero = jnp.zeros((16,), _F32)

        def zb(r, c):
            for k in range(8):
                bins[r, pl.ds(k * 16, 16)] = zero
            return c

        lax.fori_loop(0, BINR, zb, 0)

        def edge(j, c):
            d = d_v[pl.ds(j * 16, 16)]
            aa = a_v[pl.ds(j * 16, 16)]
            ri = lax.shift_right_logical(d, 7)
            ci = lax.bitwise_and(d, 127)
            m = plsc.load_gather(mxv, [ri, ci])
            w = jnp.exp(aa - m)
            a_v[pl.ds(j * 16, 16)] = w
            _bin_update(bins, tmp, d, w, False)
            return c

        lax.fori_loop(0, EW // 16, edge, 0)
        pltpu.sync_copy(a_v, w_o.at[pl.ds(wid * EW, EW)])
        pltpu.sync_copy(bins, wp_out.at[wid])

    scratch = [
        pltpu.VMEM((EW,), _F32),
        pltpu.VMEM((EW,), _I32),
        pltpu.VMEM((BINR, 128), _F32),
        pltpu.VMEM((BINR, 128), _F32),
        pltpu.VMEM((BINR, 128), _I32),
    ]
    out_t = (jax.ShapeDtypeStruct((E,), _F32),
             jax.ShapeDtypeStruct((NW, BINR, 128), _F32))
    return pl.kernel(body, out_type=out_t, mesh=_mesh(),
                     compiler_params=_SC_PARAMS,
                     scratch_types=scratch)(a, dst, mx)


def _sc_scatter1(msg, idx):
    def body(m_h, i_h, out, *rest):
        rbuf = list(rest[0:4])
        ibuf = list(rest[4:8])
        rsem = list(rest[8:12])
        isem = list(rest[12:16])
        acc = rest[16]
        cid = lax.axis_index("c")
        sid = lax.axis_index("s")
        wid = sid * 2 + cid
        ebase = wid * EW

        zero = jnp.zeros((16,), _F32)

        def zrow(r, c):
            for k in range(H // 16):
                rbuf[0][r, pl.ds(k * 16, 16)] = zero
            return c

        lax.fori_loop(0, CH, zrow, 0)

        def zacc(q, c):
            pltpu.sync_copy(rbuf[0], acc.at[pl.ds(sid * 640 + q * CH, CH)])
            return c

        lax.fori_loop(0, 8, zacc, 0)
        plsc.subcore_barrier()

        def chunk(j):
            return m_h.at[pl.ds(ebase + j * CH, CH)]

        def ichunk(j):
            return i_h.at[pl.ds(ebase + j * CH, CH)]

        def start(b, j):
            pltpu.async_copy(chunk(j), rbuf[b], rsem[b])
            pltpu.async_copy(ichunk(j), ibuf[b], isem[b])

        def do_chunk(b, j):
            pltpu.make_async_copy(chunk(j), rbuf[b], rsem[b]).wait()
            pltpu.make_async_copy(ichunk(j), ibuf[b], isem[b]).wait()
            pltpu.sync_copy(rbuf[b], acc.at[ibuf[b]], add=True)

            @pl.when(j + 4 < NCH)
            def _():
                start(b, j + 4)

        for b in range(4):
            start(b, b)

        def grp(g, c):
            for b in range(4):
                do_chunk(b, 4 * g + b)
            return c

        lax.fori_loop(0, (NCH - 1) // 4, grp, 0)
        do_chunk(0, NCH - 1)
        plsc.subcore_barrier()

        def dump(q, c):
            r0 = sid * 640 + q * CH
            pltpu.sync_copy(acc.at[pl.ds(r0, CH)], rbuf[0])
            pltpu.sync_copy(rbuf[0], out.at[cid, pl.ds(r0, CH)])
            return c

        lax.fori_loop(0, 8, dump, 0)

    scratch = (
        [pltpu.VMEM((CH, H), _F32)] * 4
        + [pltpu.VMEM((CH,), _I32)] * 4
        + [pltpu.SemaphoreType.DMA] * 8
        + [pltpu.VMEM_SHARED((NPAD, H), _F32)]
    )
    out_t = jax.ShapeDtypeStruct((2, NPAD, H), _F32)
    return pl.kernel(body, out_type=out_t, mesh=_mesh(),
                     compiler_params=_SC_PARAMS,
                     scratch_types=scratch)(msg, idx)


def _sc_scatter3(msgs, idx):
    def body(m0, m1, m2, i_h, o0, o1, o2, *rest):
        rbuf = list(rest[0:4])
        ibuf = list(rest[4:8])
        rsem = list(rest[8:12])
        isem = list(rest[12:16])
        acc = rest[16]
        cid = lax.axis_index("c")
        sid = lax.axis_index("s")
        wid = sid * 2 + cid
        ebase = wid * EW

        zero = jnp.zeros((16,), _F32)

        for m_h, out in ((m0, o0), (m1, o1), (m2, o2)):
            def zrow(r, c):
                for k in range(H // 16):
                    rbuf[0][r, pl.ds(k * 16, 16)] = zero
                return c

            lax.fori_loop(0, CH, zrow, 0)

            def zacc(q, c):
                pltpu.sync_copy(rbuf[0], acc.at[pl.ds(sid * 640 + q * CH,
                                                      CH)])
                return c

            lax.fori_loop(0, 8, zacc, 0)
            plsc.subcore_barrier()

            def chunk(j):
                return m_h.at[pl.ds(ebase + j * CH, CH)]

            def ichunk(j):
                return i_h.at[pl.ds(ebase + j * CH, CH)]

            def start(b, j):
                pltpu.async_copy(chunk(j), rbuf[b], rsem[b])
                pltpu.async_copy(ichunk(j), ibuf[b], isem[b])

            def do_chunk(b, j):
                pltpu.make_async_copy(chunk(j), rbuf[b], rsem[b]).wait()
                pltpu.make_async_copy(ichunk(j), ibuf[b], isem[b]).wait()
                pltpu.sync_copy(rbuf[b], acc.at[ibuf[b]], add=True)

                @pl.when(j + 4 < NCH)
                def _():
                    start(b, j + 4)

            for b in range(4):
                start(b, b)

            def grp(g, c):
                for b in range(4):
                    do_chunk(b, 4 * g + b)
                return c

            lax.fori_loop(0, (NCH - 1) // 4, grp, 0)
            do_chunk(0, NCH - 1)
            plsc.subcore_barrier()

            def dump(q, c):
                r0 = sid * 640 + q * CH
                pltpu.sync_copy(acc.at[pl.ds(r0, CH)], rbuf[0])
                pltpu.sync_copy(rbuf[0], out.at[cid, pl.ds(r0, CH)])
                return c

            lax.fori_loop(0, 8, dump, 0)
            plsc.subcore_barrier()

    scratch = (
        [pltpu.VMEM((CH, H), _F32)] * 4
        + [pltpu.VMEM((CH,), _I32)] * 4
        + [pltpu.SemaphoreType.DMA] * 8
        + [pltpu.VMEM_SHARED((NPAD, H), _F32)]
    )
    out_t = tuple(jax.ShapeDtypeStruct((2, NPAD, H), _F32) for _ in range(3))
    return pl.kernel(body, out_type=out_t, mesh=_mesh(), compiler_params=_SC_PARAMS,
                     scratch_types=scratch)(msgs[0], msgs[1], msgs[2], idx)


_TC_PARAMS = pltpu.CompilerParams(
    dimension_semantics=("arbitrary", "arbitrary"))


def _gelu(x):
    return 0.5 * x * (1.0 + lax.erf(x * 0.7071067811865476))


def _mlp(xin, w1, b1, w2, b2):
    y = _gelu(jnp.dot(xin, w1, preferred_element_type=_F32) + b1)
    return jnp.dot(y, w2, preferred_element_type=_F32) + b2


def _full_spec(shape):
    nd = len(shape)
    return pl.BlockSpec(shape, lambda p, i: (0,) * nd)


def _bn_phases(p, i, y, rows, stats, out_ref, post=None):
    @pl.when(jnp.logical_and(p == 0, i == 0))
    def _():
        stats[...] = jnp.zeros_like(stats)

    @pl.when(p == 0)
    def _():
        stats[0:1, :] += jnp.sum(y, axis=0, keepdims=True)
        stats[1:2, :] += jnp.sum(y * y, axis=0, keepdims=True)
        out_ref[...] = y

    @pl.when(p == 1)
    def _():
        m = stats[0:1, :] / rows
        v = stats[1:2, :] / rows - m * m
        yn = (y - m) * lax.rsqrt(v + 1e-5)
        out_ref[...] = yn
        if post is not None:
            post(yn)


def _tc_red(part, is_max):
    def body(p_ref, o_ref):
        x = p_ref[...]
        o_ref[...] = jnp.max(x, axis=0) if is_max else jnp.sum(x, axis=0)

    return pl.pallas_call(
        body,
        grid=(1,),
        in_specs=[pl.BlockSpec((NW, BINR, 128), lambda i: (0, 0, 0))],
        out_specs=pl.BlockSpec((BINR, 128), lambda i: (0, 0)),
        out_shape=jax.ShapeDtypeStruct((BINR, 128), _F32),
    )(part)


def _tc_enc(xp, w1, b1, w2, b2, rows, br):
    cin = xp.shape[1]
    nb = rows // br

    def body(x_ref, w1r, b1r, w2r, b2r, out_ref, stats):
        p = pl.program_id(0)
        i = pl.program_id(1)
        y = _mlp(x_ref[...], w1r[...], b1r[...], w2r[...], b2r[...])
        _bn_phases(p, i, y, float(rows), stats, out_ref)

    return pl.pallas_call(
        body,
        grid=(2, nb),
        in_specs=[
            pl.BlockSpec((br, cin), lambda p, i: (i, 0)),
            _full_spec((cin, H)),
            _full_spec((H,)),
            _full_spec((H, H)),
            _full_spec((H,)),
        ],
        out_specs=pl.BlockSpec((br, H), lambda p, i: (i, 0)),
        out_shape=jax.ShapeDtypeStruct((rows, H), _F32),
        scratch_shapes=[pltpu.VMEM((8, H), _F32)],
        compiler_params=_TC_PARAMS,
    )(xp, w1, b1, w2, b2)


def _tc_logits(gs, gd, eh):
    nb = E // BRE

    def body(s_ref, d_ref, e_ref, o_ref):
        o_ref[...] = jnp.sum((s_ref[...] + e_ref[...]) * d_ref[...],
                             axis=1, keepdims=True)

    spec = pl.BlockSpec((BRE, H), lambda i: (i, 0))
    return pl.pallas_call(
        body,
        grid=(nb,),
        in_specs=[spec, spec, spec],
        out_specs=pl.BlockSpec((BRE, 1), lambda i: (i, 0)),
        out_shape=jax.ShapeDtypeStruct((E, 1), _F32),
    )(gs, gd, eh)


def _tc_msgw(gs, w):
    nb = E // BRE

    def body(s_ref, w_ref, o_ref):
        o_ref[...] = s_ref[...] * w_ref[...]

    return pl.pallas_call(
        body,
        grid=(nb,),
        in_specs=[
            pl.BlockSpec((BRE, H), lambda i: (i, 0)),
            pl.BlockSpec((BRE, 1), lambda i: (i, 0)),
        ],
        out_specs=pl.BlockSpec((BRE, H), lambda i: (i, 0)),
        out_shape=jax.ShapeDtypeStruct((E, H), _F32),
    )(gs, w)


def _tc_node_update(rows_p, wsum, nh, eps, w1, b1, w2, b2, bid2, want_gh):
    nb = N // BRN

    def body(rp_ref, ws_ref, nh_ref, eps_ref, w1r, b1r, w2r, b2r, bid_ref,
             out_ref, nz_ref, *rest):
        p = pl.program_id(0)
        i = pl.program_id(1)
        rp = rp_ref[...]
        denom = ws_ref[...] + 1e-16
        nz = (rp[0] + rp[1]) / denom
        nz_ref[...] = nz
        xin = (1.0 + eps_ref[...]) * nh_ref[...] + nz
        y = _mlp(xin, w1r[...], b1r[...], w2r[...], b2r[...])
        if want_gh:
            gh_ref, stats, acc = rest

            @pl.when(jnp.logical_and(p == 1, i == 0))
            def _():
                acc[...] = jnp.zeros_like(acc)

            def post(yn):
                oh = (bid_ref[...] ==
                      lax.broadcasted_iota(_I32, (1, G), 1)).astype(_F32)
                acc[...] += lax.dot_general(
                    oh, yn, (((0,), (0,)), ((), ())),
                    preferred_element_type=_F32)
                gh_ref[...] = acc[...]
        else:
            (stats,) = rest
            post = None
        _bn_phases(p, i, y, float(N), stats, out_ref, post=post)

    out_shape = [jax.ShapeDtypeStruct((N, H), _F32),
                 jax.ShapeDtypeStruct((N, H), _F32)]
    out_specs = [pl.BlockSpec((BRN, H), lambda p, i: (i, 0)),
                 pl.BlockSpec((BRN, H), lambda p, i: (i, 0))]
    scratch = [pltpu.VMEM((8, H), _F32)]
    if want_gh:
        out_shape.append(jax.ShapeDtypeStruct((G, H), _F32))
        out_specs.append(pl.BlockSpec((G, H), lambda p, i: (0, 0)))
        scratch.append(pltpu.VMEM((G, H), _F32))

    return pl.pallas_call(
        body,
        grid=(2, nb),
        in_specs=[
            pl.BlockSpec((2, BRN, H), lambda p, i: (0, i, 0)),
            pl.BlockSpec((BRN, 1), lambda p, i: (i, 0)),
            pl.BlockSpec((BRN, H), lambda p, i: (i, 0)),
            _full_spec((H,)),
            _full_spec((H, H)),
            _full_spec((H,)),
            _full_spec((H, H)),
            _full_spec((H,)),
            pl.BlockSpec((BRN, 1), lambda p, i: (i, 0)),
        ],
        out_specs=out_specs,
        out_shape=out_shape,
        scratch_shapes=scratch,
        compiler_params=_TC_PARAMS,
    )(rows_p, wsum, nh, eps, w1, b1, w2, b2, bid2)


def _tc_edge_update(eh, zs, zd, eps, w1, b1, w2, b2):
    nb = E // BRE

    def body(eh_ref, zs_ref, zd_ref, eps_ref, w1r, b1r, w2r, b2r, out_ref,
             stats):
        p = pl.program_id(0)
        i = pl.program_id(1)
        xin = ((1.0 + eps_ref[...]) * eh_ref[...] + zs_ref[...] - zd_ref[...])
        y = _mlp(xin, w1r[...], b1r[...], w2r[...], b2r[...])
        _bn_phases(p, i, y, float(E), stats, out_ref)

    spec = pl.BlockSpec((BRE, H), lambda p, i: (i, 0))
    return pl.pallas_call(
        body,
        grid=(2, nb),
        in_specs=[spec, spec, spec, _full_spec((H,)), _full_spec((H, H)),
                  _full_spec((H,)), _full_spec((H, H)), _full_spec((H,))],
        out_specs=spec,
        out_shape=jax.ShapeDtypeStruct((E, H), _F32),
        scratch_shapes=[pltpu.VMEM((8, H), _F32)],
        compiler_params=_TC_PARAMS,
    )(eh, zs, zd, eps, w1, b1, w2, b2)


_TAGS = ("n", "e", "g")


def kernel(x, edge_attr, weights, edge_index, batch_ids):
    src = edge_index[0]
    dst = edge_index[1]
    bid2 = batch_ids.reshape(N, 1)
    xp = jnp.pad(x, ((0, 0), (0, 16 - x.shape[1])))
    ep = jnp.pad(edge_attr, ((0, 0), (0, 8 - edge_attr.shape[1])))

    def enc(arr, layers, rows, br, cin):
        w1 = jnp.pad(layers[0]["W"], ((0, cin - layers[0]["W"].shape[0]),
                                      (0, 0)))
        return _tc_enc(arr, w1, layers[0]["b"], layers[1]["W"],
                       layers[1]["b"], rows, br)

    nh = {}
    eh = {}
    for tag in _TAGS:
        nh[tag] = enc(xp, weights["nf_lin_for_" + tag], N, BRN, 16)
        eh[tag] = enc(ep, weights["ef_lin_for_" + tag], E, BRE, 8)

    final = {}
    for li in range(DEPTH):
        last = li == DEPTH - 1
        nz = {}
        for tag in _TAGS:
            p = weights["gnn_" + tag]
            nl = p["nf_lin"]
            g_s, g_d = _sc_gather_multi([nh[tag]], src, dst)
            a1 = _tc_logits(g_s, g_d, eh[tag]).reshape(E)
            mx = _tc_red(_sc_segmax1(a1, dst), True)
            w, wp = _sc_expw1(a1, dst, mx)
            wsum = _tc_red(wp, False).reshape(NPAD, 1)
            msg = _tc_msgw(g_s, w.reshape(E, 1))
            rows_p = _sc_scatter1(msg, dst)
            want_gh = tag == "g" and last
            res = _tc_node_update(rows_p, wsum, nh[tag], p["nf_eps"],
                                  nl[0]["W"], nl[0]["b"], nl[1]["W"],
                                  nl[1]["b"], bid2, want_gh)
            nh[tag], nz[tag] = res[0], res[1]
            if want_gh:
                final["g"] = res[2]
        upd = ("e",) if last else _TAGS
        for tag in upd:
            p = weights["gnn_" + tag]
            el = p["ef_lin"]
            zs, zd = _sc_gather_multi([nz[tag]], src, dst)
            eh[tag] = _tc_edge_update(eh[tag], zs, zd, p["ef_eps"],
                                      el[0]["W"], el[0]["b"],
                                      el[1]["W"], el[1]["b"])
    final["n"] = nh["n"]
    final["e"] = eh["e"]
    return (final["n"], final["e"], final["g"])


# speedup vs baseline: 1.0810x; 1.0034x over previous
import jax
import jax.numpy as jnp
from jax import lax
from jax.experimental import pallas as pl
from jax.experimental.pallas import tpu as pltpu
from jax.experimental.pallas import tpu_sc as plsc

N = 10000
E = 320000
H = 128
G = 64
DEPTH = 3

NW = 32
EW = E // NW
CH = 80
NCH = EW // CH
NG = NCH // 5
BINR = 80
NPAD = 10240
NH2 = NPAD // 2
ACCR = NH2 + 8

BRE = 1280
BRN = 2000

_F32 = jnp.float32
_I32 = jnp.int32


def _mesh():
    return plsc.VectorSubcoreMesh(core_axis_name="c", subcore_axis_name="s")


_SC_PARAMS = pltpu.CompilerParams(needs_layout_passes=False)


def _sc_gather_multi(tables, idxa, idxb):
    nt = len(tables)

    def body(*refs):
        tabs = refs[:nt]
        ia, ib = refs[nt], refs[nt + 1]
        outs = refs[nt + 2:nt + 2 + 2 * nt]
        rest = refs[nt + 2 + 2 * nt:]
        iva, ivb = rest[0], rest[1]
        rest = rest[2:]
        bufs = [list(rest[0:5]), list(rest[5:10])]
        gsem = [list(rest[10:15]), list(rest[15:20])]
        osem = [list(rest[20:25]), list(rest[25:30])]
        wid = lax.axis_index("s") * 2 + lax.axis_index("c")
        ebase = wid * EW
        pltpu.sync_copy(ia.at[pl.ds(ebase, EW)], iva)
        pltpu.sync_copy(ib.at[pl.ds(ebase, EW)], ivb)

        def stream(tab, iv, out):
            def chunk(j):
                return out.at[pl.ds(ebase + j * CH, CH)]

            for b in range(5):
                pltpu.async_copy(tab.at[iv.at[pl.ds(b * CH, CH)]],
                                 bufs[0][b], gsem[0][b])

            def emit(bank, g):
                ob_ = 1 - bank
                for b in range(5):
                    @pl.when(g > 0)
                    def _():
                        pltpu.make_async_copy(
                            bufs[ob_][b], chunk(0), osem[ob_][b]).wait()

                    @pl.when(g < NG - 1)
                    def _():
                        pltpu.async_copy(
                            tab.at[iv.at[pl.ds((5 * (g + 1) + b) * CH, CH)]],
                            bufs[ob_][b], gsem[ob_][b])
                for b in range(5):
                    pltpu.make_async_copy(
                        tab.at[iv.at[pl.ds((5 * g + b) * CH, CH)]],
                        bufs[bank][b], gsem[bank][b]).wait()
                    pltpu.async_copy(bufs[bank][b], chunk(5 * g + b),
                                     osem[bank][b])

            def grp(g, c):
                @pl.when(g % 2 == 0)
                def _():
                    emit(0, g)

                @pl.when(g % 2 == 1)
                def _():
                    emit(1, g)
                return c

            lax.fori_loop(0, NG, grp, 0)
            bank = (NG - 1) % 2
            for b in range(5):
                pltpu.make_async_copy(bufs[bank][b], chunk(0),
                                      osem[bank][b]).wait()

        for t in range(nt):
            stream(tabs[t], iva, outs[2 * t])
            stream(tabs[t], ivb, outs[2 * t + 1])

    scratch = (
        [pltpu.VMEM((EW,), _I32)] * 2
        + [pltpu.VMEM((CH, H), _F32)] * 10
        + [pltpu.SemaphoreType.DMA] * 20
    )
    out_t = tuple(jax.ShapeDtypeStruct((E, H), _F32) for _ in range(2 * nt))
    return pl.kernel(body, out_type=out_t, mesh=_mesh(),
                     compiler_params=_SC_PARAMS,
                     scratch_types=scratch)(*tables, idxa, idxb)


def _bin_update(bins, tmp, d, vv, is_max):
    ri = lax.shift_right_logical(d, 7)
    ci = lax.bitwise_and(d, 127)
    iot = lax.iota(_I32, 16)
    ones = jnp.ones((16,), _I32)
    zeros = jnp.zeros((16,), _I32)

    def round_(rem):
        remb = rem != 0
        plsc.store_scatter(tmp, [ri, ci], iot, mask=remb)
        back = plsc.load_gather(tmp, [ri, ci])
        win = jnp.logical_and(remb, back == iot)
        cur = plsc.load_gather(bins, [ri, ci])
        upd = jnp.maximum(cur, vv) if is_max else cur + vv
        plsc.store_scatter(bins, [ri, ci], upd, mask=win)
        return jnp.where(jnp.logical_and(remb, jnp.logical_not(win)),
                         ones, zeros)

    rem0 = round_(ones)

    @pl.when(jnp.any(rem0 != 0))
    def _():
        def rounds(r, rm):
            return round_(rm)

        lax.fori_loop(0, 15, rounds, rem0)


def _sc_segmax3(a0, a1, a2, dst):
    def body(a0_h, a1_h, a2_h, d_h, out, av0, av1, av2, d_v, b0, b1, b2,
             tmp):
        wid = lax.axis_index("s") * 2 + lax.axis_index("c")
        avs = (av0, av1, av2)
        bins = (b0, b1, b2)
        for ah, av in zip((a0_h, a1_h, a2_h), avs):
            pltpu.sync_copy(ah.at[pl.ds(wid * EW, EW)], av)
        pltpu.sync_copy(d_h.at[pl.ds(wid * EW, EW)], d_v)

        neg = jnp.full((16,), -1e30, _F32)

        def zb(r, c):
            for bb in bins:
                for k in range(8):
                    bb[r, pl.ds(k * 16, 16)] = neg
            return c

        lax.fori_loop(0, BINR, zb, 0)

        def edge(j, c):
            d = d_v[pl.ds(j * 16, 16)]
            for av, bb in zip(avs, bins):
                v = av[pl.ds(j * 16, 16)]
                _bin_update(bb, tmp, d, v, True)
            return c

        lax.fori_loop(0, EW // 16, edge, 0)
        for t, bb in enumerate(bins):
            pltpu.sync_copy(bb, out.at[t, wid])

    scratch = [
        pltpu.VMEM((EW,), _F32),
        pltpu.VMEM((EW,), _F32),
        pltpu.VMEM((EW,), _F32),
        pltpu.VMEM((EW,), _I32),
        pltpu.VMEM((BINR, 128), _F32),
        pltpu.VMEM((BINR, 128), _F32),
        pltpu.VMEM((BINR, 128), _F32),
        pltpu.VMEM((BINR, 128), _I32),
    ]
    out_t = jax.ShapeDtypeStruct((3, NW, BINR, 128), _F32)
    return pl.kernel(body, out_type=out_t, mesh=_mesh(),
                     compiler_params=_SC_PARAMS,
                     scratch_types=scratch)(a0, a1, a2, dst)


def _sc_expw3(a0, a1, a2, dst, mx3):
    def body(a0_h, a1_h, a2_h, d_h, mx_h, w0_o, w1_o, w2_o, wp_out,
             av0, av1, av2, d_v, mxv, b0, b1, b2, tmp):
        wid = lax.axis_index("s") * 2 + lax.axis_index("c")
        avs = (av0, av1, av2)
        bins = (b0, b1, b2)
        for ah, av in zip((a0_h, a1_h, a2_h), avs):
            pltpu.sync_copy(ah.at[pl.ds(wid * EW, EW)], av)
        pltpu.sync_copy(d_h.at[pl.ds(wid * EW, EW)], d_v)

        zero = jnp.zeros((16,), _F32)

        def zb(r, c):
            for bb in bins:
                for k in range(8):
                    bb[r, pl.ds(k * 16, 16)] = zero
            return c

        lax.fori_loop(0, BINR, zb, 0)

        for t, (av, bb) in enumerate(zip(avs, bins)):
            pltpu.sync_copy(mx_h.at[t], mxv)

            def edge(j, c):
                d = d_v[pl.ds(j * 16, 16)]
                aa = av[pl.ds(j * 16, 16)]
                ri = lax.shift_right_logical(d, 7)
                ci = lax.bitwise_and(d, 127)
                m = plsc.load_gather(mxv, [ri, ci])
                w = jnp.exp(aa - m)
                av[pl.ds(j * 16, 16)] = w
                _bin_update(bb, tmp, d, w, False)
                return c

            lax.fori_loop(0, EW // 16, edge, 0)

        for t, (av, wo) in enumerate(zip(avs, (w0_o, w1_o, w2_o))):
            pltpu.sync_copy(av, wo.at[pl.ds(wid * EW, EW)])
        for t, bb in enumerate(bins):
            pltpu.sync_copy(bb, wp_out.at[t, wid])

    scratch = [
        pltpu.VMEM((EW,), _F32),
        pltpu.VMEM((EW,), _F32),
        pltpu.VMEM((EW,), _F32),
        pltpu.VMEM((EW,), _I32),
        pltpu.VMEM((BINR, 128), _F32),
        pltpu.VMEM((BINR, 128), _F32),
        pltpu.VMEM((BINR, 128), _F32),
        pltpu.VMEM((BINR, 128), _F32),
        pltpu.VMEM((BINR, 128), _I32),
    ]
    out_t = (jax.ShapeDtypeStruct((E,), _F32),
             jax.ShapeDtypeStruct((E,), _F32),
             jax.ShapeDtypeStruct((E,), _F32),
             jax.ShapeDtypeStruct((3, NW, BINR, 128), _F32))
    return pl.kernel(body, out_type=out_t, mesh=_mesh(),
                     compiler_params=_SC_PARAMS,
                     scratch_types=scratch)(a0, a1, a2, dst, mx3)


def _sc_segmax1(a, dst):
    def body(a_h, d_h, out, a_v, d_v, bins, tmp):
        wid = lax.axis_index("s") * 2 + lax.axis_index("c")
        pltpu.sync_copy(a_h.at[pl.ds(wid * EW, EW)], a_v)
        pltpu.sync_copy(d_h.at[pl.ds(wid * EW, EW)], d_v)

        neg = jnp.full((16,), -1e30, _F32)

        def zb(r, c):
            for k in range(8):
                bins[r, pl.ds(k * 16, 16)] = neg
            return c

        lax.fori_loop(0, BINR, zb, 0)

        def edge(j, c):
            d = d_v[pl.ds(j * 16, 16)]
            v = a_v[pl.ds(j * 16, 16)]
            _bin_update(bins, tmp, d, v, True)
            return c

        lax.fori_loop(0, EW // 16, edge, 0)
        pltpu.sync_copy(bins, out.at[wid])

    scratch = [
        pltpu.VMEM((EW,), _F32),
        pltpu.VMEM((EW,), _I32),
        pltpu.VMEM((BINR, 128), _F32),
        pltpu.VMEM((BINR, 128), _I32),
    ]
    out_t = jax.ShapeDtypeStruct((NW, BINR, 128), _F32)
    return pl.kernel(body, out_type=out_t, mesh=_mesh(),
                     compiler_params=_SC_PARAMS,
                     scratch_types=scratch)(a, dst)


def _sc_expw1(a, dst, mx):
    def body(a_h, d_h, mx_h, w_o, wp_out, a_v, d_v, mxv, bins):
        wid = lax.axis_index("s") * 2 + lax.axis_index("c")
        pltpu.sync_copy(a_h.at[pl.ds(wid * EW, EW)], a_v)
        pltpu.sync_copy(d_h.at[pl.ds(wid * EW, EW)], d_v)
        pltpu.sync_copy(mx_h, mxv)

        zero = jnp.zeros((16,), _F32)

        def zb(r, c):
            for k in range(8):
                bins[r, pl.ds(k * 16, 16)] = zero
            return c

        lax.fori_loop(0, BINR, zb, 0)

        def edge(j, c):
            d = d_v[pl.ds(j * 16, 16)]
            aa = a_v[pl.ds(j * 16, 16)]
            ri = lax.shift_right_logical(d, 7)
            ci = lax.bitwise_and(d, 127)
            m = plsc.load_gather(mxv, [ri, ci])
            w = jnp.exp(aa - m)
            a_v[pl.ds(j * 16, 16)] = w
            plsc.addupdate_scatter(bins, [ri, ci], w)
            return c

        lax.fori_loop(0, EW // 16, edge, 0)
        pltpu.sync_copy(a_v, w_o.at[pl.ds(wid * EW, EW)])
        pltpu.sync_copy(bins, wp_out.at[wid])

    scratch = [
        pltpu.VMEM((EW,), _F32),
        pltpu.VMEM((EW,), _I32),
        pltpu.VMEM((BINR, 128), _F32),
        pltpu.VMEM((BINR, 128), _F32),
    ]
    out_t = (jax.ShapeDtypeStruct((E,), _F32),
             jax.ShapeDtypeStruct((NW, BINR, 128), _F32))
    return pl.kernel(body, out_type=out_t, mesh=_mesh(),
                     compiler_params=_SC_PARAMS,
                     scratch_types=scratch)(a, dst, mx)


def _sc_scatter1(msg, idx):
    def body(m_h, i_h, out, *rest):
        rbuf = list(rest[0:4])
        ibuf = list(rest[4:8])
        rsem = list(rest[8:12])
        isem = list(rest[12:16])
        acc = rest[16]
        cid = lax.axis_index("c")
        sid = lax.axis_index("s")
        wid = sid * 2 + cid
        ebase = wid * EW

        zero = jnp.zeros((16,), _F32)

        def zrow(r, c):
            for k in range(H // 16):
                rbuf[0][r, pl.ds(k * 16, 16)] = zero
            return c

        lax.fori_loop(0, CH, zrow, 0)

        def zacc(q, c):
            pltpu.sync_copy(rbuf[0], acc.at[pl.ds(sid * 640 + q * CH, CH)])
            return c

        lax.fori_loop(0, 8, zacc, 0)
        plsc.subcore_barrier()

        def chunk(j):
            return m_h.at[pl.ds(ebase + j * CH, CH)]

        def ichunk(j):
            return i_h.at[pl.ds(ebase + j * CH, CH)]

        def start(b, j):
            pltpu.async_copy(chunk(j), rbuf[b], rsem[b])
            pltpu.async_copy(ichunk(j), ibuf[b], isem[b])

        def do_chunk(b, j):
            pltpu.make_async_copy(chunk(j), rbuf[b], rsem[b]).wait()
            pltpu.make_async_copy(ichunk(j), ibuf[b], isem[b]).wait()
            pltpu.sync_copy(rbuf[b], acc.at[ibuf[b]], add=True)

            @pl.when(j + 4 < NCH)
            def _():
                start(b, j + 4)

        for b in range(4):
            start(b, b)

        def grp(g, c):
            for b in range(4):
                do_chunk(b, 4 * g + b)
            return c

        lax.fori_loop(0, (NCH - 1) // 4, grp, 0)
        do_chunk(0, NCH - 1)
        plsc.subcore_barrier()

        def dump(q, c):
            r0 = sid * 640 + q * CH
            pltpu.sync_copy(acc.at[pl.ds(r0, CH)], rbuf[0])
            pltpu.sync_copy(rbuf[0], out.at[cid, pl.ds(r0, CH)])
            return c

        lax.fori_loop(0, 8, dump, 0)

    scratch = (
        [pltpu.VMEM((CH, H), _F32)] * 4
        + [pltpu.VMEM((CH,), _I32)] * 4
        + [pltpu.SemaphoreType.DMA] * 8
        + [pltpu.VMEM_SHARED((NPAD, H), _F32)]
    )
    out_t = jax.ShapeDtypeStruct((2, NPAD, H), _F32)
    return pl.kernel(body, out_type=out_t, mesh=_mesh(),
                     compiler_params=_SC_PARAMS,
                     scratch_types=scratch)(msg, idx)


def _sc_scatter3(msgs, idx):
    def body(m0, m1, m2, i_h, o0, o1, o2, *rest):
        rbuf = list(rest[0:4])
        ibuf = list(rest[4:8])
        rsem = list(rest[8:12])
        isem = list(rest[12:16])
        acc = rest[16]
        cid = lax.axis_index("c")
        sid = lax.axis_index("s")
        wid = sid * 2 + cid
        ebase = wid * EW

        zero = jnp.zeros((16,), _F32)

        for m_h, out in ((m0, o0), (m1, o1), (m2, o2)):
            def zrow(r, c):
                for k in range(H // 16):
                    rbuf[0][r, pl.ds(k * 16, 16)] = zero
                return c

            lax.fori_loop(0, CH, zrow, 0)

            def zacc(q, c):
                pltpu.sync_copy(rbuf[0], acc.at[pl.ds(sid * 640 + q * CH,
                                                      CH)])
                return c

            lax.fori_loop(0, 8, zacc, 0)
            plsc.subcore_barrier()

            def chunk(j):
                return m_h.at[pl.ds(ebase + j * CH, CH)]

            def ichunk(j):
                return i_h.at[pl.ds(ebase + j * CH, CH)]

            def start(b, j):
                pltpu.async_copy(chunk(j), rbuf[b], rsem[b])
                pltpu.async_copy(ichunk(j), ibuf[b], isem[b])

            def do_chunk(b, j):
                pltpu.make_async_copy(chunk(j), rbuf[b], rsem[b]).wait()
                pltpu.make_async_copy(ichunk(j), ibuf[b], isem[b]).wait()
                pltpu.sync_copy(rbuf[b], acc.at[ibuf[b]], add=True)

                @pl.when(j + 4 < NCH)
                def _():
                    start(b, j + 4)

            for b in range(4):
                start(b, b)

            def grp(g, c):
                for b in range(4):
                    do_chunk(b, 4 * g + b)
                return c

            lax.fori_loop(0, (NCH - 1) // 4, grp, 0)
            do_chunk(0, NCH - 1)
            plsc.subcore_barrier()

            def dump(q, c):
                r0 = sid * 640 + q * CH
                pltpu.sync_copy(acc.at[pl.ds(r0, CH)], rbuf[0])
                pltpu.sync_copy(rbuf[0], out.at[cid, pl.ds(r0, CH)])
                return c

            lax.fori_loop(0, 8, dump, 0)
            plsc.subcore_barrier()

    scratch = (
        [pltpu.VMEM((CH, H), _F32)] * 4
        + [pltpu.VMEM((CH,), _I32)] * 4
        + [pltpu.SemaphoreType.DMA] * 8
        + [pltpu.VMEM_SHARED((NPAD, H), _F32)]
    )
    out_t = tuple(jax.ShapeDtypeStruct((2, NPAD, H), _F32) for _ in range(3))
    return pl.kernel(body, out_type=out_t, mesh=_mesh(), compiler_params=_SC_PARAMS,
                     scratch_types=scratch)(msgs[0], msgs[1], msgs[2], idx)


_TC_PARAMS = pltpu.CompilerParams(
    dimension_semantics=("arbitrary", "arbitrary"))


def _gelu(x):
    return 0.5 * x * (1.0 + lax.erf(x * 0.7071067811865476))


def _mlp(xin, w1, b1, w2, b2):
    y = _gelu(jnp.dot(xin, w1, preferred_element_type=_F32) + b1)
    return jnp.dot(y, w2, preferred_element_type=_F32) + b2


def _full_spec(shape):
    nd = len(shape)
    return pl.BlockSpec(shape, lambda p, i: (0,) * nd)


def _bn_phases(p, i, y, rows, stats, out_ref, post=None):
    @pl.when(jnp.logical_and(p == 0, i == 0))
    def _():
        stats[...] = jnp.zeros_like(stats)

    @pl.when(p == 0)
    def _():
        stats[0:1, :] += jnp.sum(y, axis=0, keepdims=True)
        stats[1:2, :] += jnp.sum(y * y, axis=0, keepdims=True)
        out_ref[...] = y

    @pl.when(p == 1)
    def _():
        m = stats[0:1, :] / rows
        v = stats[1:2, :] / rows - m * m
        yn = (y - m) * lax.rsqrt(v + 1e-5)
        out_ref[...] = yn
        if post is not None:
            post(yn)


def _tc_red(part, is_max):
    def body(p_ref, o_ref):
        x = p_ref[...]
        o_ref[...] = jnp.max(x, axis=0) if is_max else jnp.sum(x, axis=0)

    return pl.pallas_call(
        body,
        grid=(1,),
        in_specs=[pl.BlockSpec((NW, BINR, 128), lambda i: (0, 0, 0))],
        out_specs=pl.BlockSpec((BINR, 128), lambda i: (0, 0)),
        out_shape=jax.ShapeDtypeStruct((BINR, 128), _F32),
    )(part)


def _tc_enc(xp, w1, b1, w2, b2, rows, br):
    cin = xp.shape[1]
    nb = rows // br

    def body(x_ref, w1r, b1r, w2r, b2r, out_ref, stats):
        p = pl.program_id(0)
        i = pl.program_id(1)
        y = _mlp(x_ref[...], w1r[...], b1r[...], w2r[...], b2r[...])
        _bn_phases(p, i, y, float(rows), stats, out_ref)

    return pl.pallas_call(
        body,
        grid=(2, nb),
        in_specs=[
            pl.BlockSpec((br, cin), lambda p, i: (i, 0)),
            _full_spec((cin, H)),
            _full_spec((H,)),
            _full_spec((H, H)),
            _full_spec((H,)),
        ],
        out_specs=pl.BlockSpec((br, H), lambda p, i: (i, 0)),
        out_shape=jax.ShapeDtypeStruct((rows, H), _F32),
        scratch_shapes=[pltpu.VMEM((8, H), _F32)],
        compiler_params=_TC_PARAMS,
    )(xp, w1, b1, w2, b2)


def _tc_logits(gs, gd, eh):
    nb = E // BRE

    def body(s_ref, d_ref, e_ref, o_ref):
        o_ref[...] = jnp.sum((s_ref[...] + e_ref[...]) * d_ref[...],
                             axis=1, keepdims=True)

    spec = pl.BlockSpec((BRE, H), lambda i: (i, 0))
    return pl.pallas_call(
        body,
        grid=(nb,),
        in_specs=[spec, spec, spec],
        out_specs=pl.BlockSpec((BRE, 1), lambda i: (i, 0)),
        out_shape=jax.ShapeDtypeStruct((E, 1), _F32),
    )(gs, gd, eh)


def _tc_msgw(gs, w):
    nb = E // BRE

    def body(s_ref, w_ref, o_ref):
        o_ref[...] = s_ref[...] * w_ref[...]

    return pl.pallas_call(
        body,
        grid=(nb,),
        in_specs=[
            pl.BlockSpec((BRE, H), lambda i: (i, 0)),
            pl.BlockSpec((BRE, 1), lambda i: (i, 0)),
        ],
        out_specs=pl.BlockSpec((BRE, H), lambda i: (i, 0)),
        out_shape=jax.ShapeDtypeStruct((E, H), _F32),
    )(gs, w)


def _tc_node_update(rows_p, wsum, nh, eps, w1, b1, w2, b2, bid2, want_gh):
    nb = N // BRN

    def body(rp_ref, ws_ref, nh_ref, eps_ref, w1r, b1r, w2r, b2r, bid_ref,
             out_ref, nz_ref, *rest):
        p = pl.program_id(0)
        i = pl.program_id(1)
        rp = rp_ref[...]
        denom = ws_ref[...] + 1e-16
        nz = (rp[0] + rp[1]) / denom
        nz_ref[...] = nz
        xin = (1.0 + eps_ref[...]) * nh_ref[...] + nz
        y = _mlp(xin, w1r[...], b1r[...], w2r[...], b2r[...])
        if want_gh:
            gh_ref, stats, acc = rest

            @pl.when(jnp.logical_and(p == 1, i == 0))
            def _():
                acc[...] = jnp.zeros_like(acc)

            def post(yn):
                oh = (bid_ref[...] ==
                      lax.broadcasted_iota(_I32, (1, G), 1)).astype(_F32)
                acc[...] += lax.dot_general(
                    oh, yn, (((0,), (0,)), ((), ())),
                    preferred_element_type=_F32)
                gh_ref[...] = acc[...]
        else:
            (stats,) = rest
            post = None
        _bn_phases(p, i, y, float(N), stats, out_ref, post=post)

    out_shape = [jax.ShapeDtypeStruct((N, H), _F32),
                 jax.ShapeDtypeStruct((N, H), _F32)]
    out_specs = [pl.BlockSpec((BRN, H), lambda p, i: (i, 0)),
                 pl.BlockSpec((BRN, H), lambda p, i: (i, 0))]
    scratch = [pltpu.VMEM((8, H), _F32)]
    if want_gh:
        out_shape.append(jax.ShapeDtypeStruct((G, H), _F32))
        out_specs.append(pl.BlockSpec((G, H), lambda p, i: (0, 0)))
        scratch.append(pltpu.VMEM((G, H), _F32))

    return pl.pallas_call(
        body,
        grid=(2, nb),
        in_specs=[
            pl.BlockSpec((2, BRN, H), lambda p, i: (0, i, 0)),
            pl.BlockSpec((BRN, 1), lambda p, i: (i, 0)),
            pl.BlockSpec((BRN, H), lambda p, i: (i, 0)),
            _full_spec((H,)),
            _full_spec((H, H)),
            _full_spec((H,)),
            _full_spec((H, H)),
            _full_spec((H,)),
            pl.BlockSpec((BRN, 1), lambda p, i: (i, 0)),
        ],
        out_specs=out_specs,
        out_shape=out_shape,
        scratch_shapes=scratch,
        compiler_params=_TC_PARAMS,
    )(rows_p, wsum, nh, eps, w1, b1, w2, b2, bid2)


def _tc_edge_update(eh, zs, zd, eps, w1, b1, w2, b2):
    nb = E // BRE

    def body(eh_ref, zs_ref, zd_ref, eps_ref, w1r, b1r, w2r, b2r, out_ref,
             stats):
        p = pl.program_id(0)
        i = pl.program_id(1)
        xin = ((1.0 + eps_ref[...]) * eh_ref[...] + zs_ref[...] - zd_ref[...])
        y = _mlp(xin, w1r[...], b1r[...], w2r[...], b2r[...])
        _bn_phases(p, i, y, float(E), stats, out_ref)

    spec = pl.BlockSpec((BRE, H), lambda p, i: (i, 0))
    return pl.pallas_call(
        body,
        grid=(2, nb),
        in_specs=[spec, spec, spec, _full_spec((H,)), _full_spec((H, H)),
                  _full_spec((H,)), _full_spec((H, H)), _full_spec((H,))],
        out_specs=spec,
        out_shape=jax.ShapeDtypeStruct((E, H), _F32),
        scratch_shapes=[pltpu.VMEM((8, H), _F32)],
        compiler_params=_TC_PARAMS,
    )(eh, zs, zd, eps, w1, b1, w2, b2)


_TAGS = ("n", "e", "g")


def kernel(x, edge_attr, weights, edge_index, batch_ids):
    src = edge_index[0]
    dst = edge_index[1]
    bid2 = batch_ids.reshape(N, 1)
    xp = jnp.pad(x, ((0, 0), (0, 16 - x.shape[1])))
    ep = jnp.pad(edge_attr, ((0, 0), (0, 8 - edge_attr.shape[1])))

    def enc(arr, layers, rows, br, cin):
        w1 = jnp.pad(layers[0]["W"], ((0, cin - layers[0]["W"].shape[0]),
                                      (0, 0)))
        return _tc_enc(arr, w1, layers[0]["b"], layers[1]["W"],
                       layers[1]["b"], rows, br)

    nh = {}
    eh = {}
    for tag in _TAGS:
        nh[tag] = enc(xp, weights["nf_lin_for_" + tag], N, BRN, 16)
        eh[tag] = enc(ep, weights["ef_lin_for_" + tag], E, BRE, 8)

    final = {}
    for li in range(DEPTH):
        last = li == DEPTH - 1
        nz = {}
        for tag in _TAGS:
            p = weights["gnn_" + tag]
            nl = p["nf_lin"]
            g_s, g_d = _sc_gather_multi([nh[tag]], src, dst)
            a1 = _tc_logits(g_s, g_d, eh[tag]).reshape(E)
            mx = _tc_red(_sc_segmax1(a1, dst), True)
            w, wp = _sc_expw1(a1, dst, mx)
            wsum = _tc_red(wp, False).reshape(NPAD, 1)
            msg = _tc_msgw(g_s, w.reshape(E, 1))
            rows_p = _sc_scatter1(msg, dst)
            want_gh = tag == "g" and last
            res = _tc_node_update(rows_p, wsum, nh[tag], p["nf_eps"],
                                  nl[0]["W"], nl[0]["b"], nl[1]["W"],
                                  nl[1]["b"], bid2, want_gh)
            nh[tag], nz[tag] = res[0], res[1]
            if want_gh:
                final["g"] = res[2]
        upd = ("e",) if last else _TAGS
        for tag in upd:
            p = weights["gnn_" + tag]
            el = p["ef_lin"]
            zs, zd = _sc_gather_multi([nz[tag]], src, dst)
            eh[tag] = _tc_edge_update(eh[tag], zs, zd, p["ef_eps"],
                                      el[0]["W"], el[0]["b"],
                                      el[1]["W"], el[1]["b"])
    final["n"] = nh["n"]
    final["e"] = eh["e"]
    return (final["n"], final["e"], final["g"])


# speedup vs baseline: 1.0815x; 1.0005x over previous
import jax
import jax.numpy as jnp
from jax import lax
from jax.experimental import pallas as pl
from jax.experimental.pallas import tpu as pltpu
from jax.experimental.pallas import tpu_sc as plsc

N = 10000
E = 320000
H = 128
G = 64
DEPTH = 3

NW = 32
EW = E // NW
CH = 80
NCH = EW // CH
NG = NCH // 5
BINR = 80
NPAD = 10240
NH2 = NPAD // 2
ACCR = NH2 + 8

BRE = 1280
BRN = 2000

_F32 = jnp.float32
_I32 = jnp.int32


def _mesh():
    return plsc.VectorSubcoreMesh(core_axis_name="c", subcore_axis_name="s")


_SC_PARAMS = pltpu.CompilerParams(needs_layout_passes=False)


def _sc_gather_multi(tables, idxa, idxb):
    nt = len(tables)

    def body(*refs):
        tabs = refs[:nt]
        ia, ib = refs[nt], refs[nt + 1]
        outs = refs[nt + 2:nt + 2 + 2 * nt]
        rest = refs[nt + 2 + 2 * nt:]
        iva, ivb = rest[0], rest[1]
        rest = rest[2:]
        bufs = [list(rest[0:5]), list(rest[5:10])]
        gsem = [list(rest[10:15]), list(rest[15:20])]
        osem = [list(rest[20:25]), list(rest[25:30])]
        wid = lax.axis_index("s") * 2 + lax.axis_index("c")
        ebase = wid * EW
        pltpu.sync_copy(ia.at[pl.ds(ebase, EW)], iva)
        pltpu.sync_copy(ib.at[pl.ds(ebase, EW)], ivb)

        def stream(tab, iv, out):
            def chunk(j):
                return out.at[pl.ds(ebase + j * CH, CH)]

            for b in range(5):
                pltpu.async_copy(tab.at[iv.at[pl.ds(b * CH, CH)]],
                                 bufs[0][b], gsem[0][b])

            def emit(bank, g):
                ob_ = 1 - bank
                for b in range(5):
                    @pl.when(g > 0)
                    def _():
                        pltpu.make_async_copy(
                            bufs[ob_][b], chunk(0), osem[ob_][b]).wait()

                    @pl.when(g < NG - 1)
                    def _():
                        pltpu.async_copy(
                            tab.at[iv.at[pl.ds((5 * (g + 1) + b) * CH, CH)]],
                            bufs[ob_][b], gsem[ob_][b])
                for b in range(5):
                    pltpu.make_async_copy(
                        tab.at[iv.at[pl.ds((5 * g + b) * CH, CH)]],
                        bufs[bank][b], gsem[bank][b]).wait()
                    pltpu.async_copy(bufs[bank][b], chunk(5 * g + b),
                                     osem[bank][b])

            def grp(g, c):
                @pl.when(g % 2 == 0)
                def _():
                    emit(0, g)

                @pl.when(g % 2 == 1)
                def _():
                    emit(1, g)
                return c

            lax.fori_loop(0, NG, grp, 0)
            bank = (NG - 1) % 2
            for b in range(5):
                pltpu.make_async_copy(bufs[bank][b], chunk(0),
                                      osem[bank][b]).wait()

        for t in range(nt):
            stream(tabs[t], iva, outs[2 * t])
            stream(tabs[t], ivb, outs[2 * t + 1])

    scratch = (
        [pltpu.VMEM((EW,), _I32)] * 2
        + [pltpu.VMEM((CH, H), _F32)] * 10
        + [pltpu.SemaphoreType.DMA] * 20
    )
    out_t = tuple(jax.ShapeDtypeStruct((E, H), _F32) for _ in range(2 * nt))
    return pl.kernel(body, out_type=out_t, mesh=_mesh(),
                     compiler_params=_SC_PARAMS,
                     scratch_types=scratch)(*tables, idxa, idxb)


def _bin_update(bins, tmp, d, vv, is_max):
    ri = lax.shift_right_logical(d, 7)
    ci = lax.bitwise_and(d, 127)
    iot = lax.iota(_I32, 16)
    ones = jnp.ones((16,), _I32)
    zeros = jnp.zeros((16,), _I32)

    def round_(rem):
        remb = rem != 0
        plsc.store_scatter(tmp, [ri, ci], iot, mask=remb)
        back = plsc.load_gather(tmp, [ri, ci])
        win = jnp.logical_and(remb, back == iot)
        cur = plsc.load_gather(bins, [ri, ci])
        upd = jnp.maximum(cur, vv) if is_max else cur + vv
        plsc.store_scatter(bins, [ri, ci], upd, mask=win)
        return jnp.where(jnp.logical_and(remb, jnp.logical_not(win)),
                         ones, zeros)

    rem0 = round_(ones)

    @pl.when(jnp.any(rem0 != 0))
    def _():
        def rounds(r, rm):
            return round_(rm)

        lax.fori_loop(0, 15, rounds, rem0)


def _sc_segmax3(a0, a1, a2, dst):
    def body(a0_h, a1_h, a2_h, d_h, out, av0, av1, av2, d_v, b0, b1, b2,
             tmp):
        wid = lax.axis_index("s") * 2 + lax.axis_index("c")
        avs = (av0, av1, av2)
        bins = (b0, b1, b2)
        for ah, av in zip((a0_h, a1_h, a2_h), avs):
            pltpu.sync_copy(ah.at[pl.ds(wid * EW, EW)], av)
        pltpu.sync_copy(d_h.at[pl.ds(wid * EW, EW)], d_v)

        neg = jnp.full((16,), -1e30, _F32)

        def zb(r, c):
            for bb in bins:
                for k in range(8):
                    bb[r, pl.ds(k * 16, 16)] = neg
            return c

        lax.fori_loop(0, BINR, zb, 0)

        def edge(j, c):
            d = d_v[pl.ds(j * 16, 16)]
            for av, bb in zip(avs, bins):
                v = av[pl.ds(j * 16, 16)]
                _bin_update(bb, tmp, d, v, True)
            return c

        lax.fori_loop(0, EW // 16, edge, 0)
        for t, bb in enumerate(bins):
            pltpu.sync_copy(bb, out.at[t, wid])

    scratch = [
        pltpu.VMEM((EW,), _F32),
        pltpu.VMEM((EW,), _F32),
        pltpu.VMEM((EW,), _F32),
        pltpu.VMEM((EW,), _I32),
        pltpu.VMEM((BINR, 128), _F32),
        pltpu.VMEM((BINR, 128), _F32),
        pltpu.VMEM((BINR, 128), _F32),
        pltpu.VMEM((BINR, 128), _I32),
    ]
    out_t = jax.ShapeDtypeStruct((3, NW, BINR, 128), _F32)
    return pl.kernel(body, out_type=out_t, mesh=_mesh(),
                     compiler_params=_SC_PARAMS,
                     scratch_types=scratch)(a0, a1, a2, dst)


def _sc_expw3(a0, a1, a2, dst, mx3):
    def body(a0_h, a1_h, a2_h, d_h, mx_h, w0_o, w1_o, w2_o, wp_out,
             av0, av1, av2, d_v, mxv, b0, b1, b2, tmp):
        wid = lax.axis_index("s") * 2 + lax.axis_index("c")
        avs = (av0, av1, av2)
        bins = (b0, b1, b2)
        for ah, av in zip((a0_h, a1_h, a2_h), avs):
            pltpu.sync_copy(ah.at[pl.ds(wid * EW, EW)], av)
        pltpu.sync_copy(d_h.at[pl.ds(wid * EW, EW)], d_v)

        zero = jnp.zeros((16,), _F32)

        def zb(r, c):
            for bb in bins:
                for k in range(8):
                    bb[r, pl.ds(k * 16, 16)] = zero
            return c

        lax.fori_loop(0, BINR, zb, 0)

        for t, (av, bb) in enumerate(zip(avs, bins)):
            pltpu.sync_copy(mx_h.at[t], mxv)

            def edge(j, c):
                d = d_v[pl.ds(j * 16, 16)]
                aa = av[pl.ds(j * 16, 16)]
                ri = lax.shift_right_logical(d, 7)
                ci = lax.bitwise_and(d, 127)
                m = plsc.load_gather(mxv, [ri, ci])
                w = jnp.exp(aa - m)
                av[pl.ds(j * 16, 16)] = w
                _bin_update(bb, tmp, d, w, False)
                return c

            lax.fori_loop(0, EW // 16, edge, 0)

        for t, (av, wo) in enumerate(zip(avs, (w0_o, w1_o, w2_o))):
            pltpu.sync_copy(av, wo.at[pl.ds(wid * EW, EW)])
        for t, bb in enumerate(bins):
            pltpu.sync_copy(bb, wp_out.at[t, wid])

    scratch = [
        pltpu.VMEM((EW,), _F32),
        pltpu.VMEM((EW,), _F32),
        pltpu.VMEM((EW,), _F32),
        pltpu.VMEM((EW,), _I32),
        pltpu.VMEM((BINR, 128), _F32),
        pltpu.VMEM((BINR, 128), _F32),
        pltpu.VMEM((BINR, 128), _F32),
        pltpu.VMEM((BINR, 128), _F32),
        pltpu.VMEM((BINR, 128), _I32),
    ]
    out_t = (jax.ShapeDtypeStruct((E,), _F32),
             jax.ShapeDtypeStruct((E,), _F32),
             jax.ShapeDtypeStruct((E,), _F32),
             jax.ShapeDtypeStruct((3, NW, BINR, 128), _F32))
    return pl.kernel(body, out_type=out_t, mesh=_mesh(),
                     compiler_params=_SC_PARAMS,
                     scratch_types=scratch)(a0, a1, a2, dst, mx3)


def _sc_segmax1(a, dst):
    def body(a_h, d_h, out, a_v, d_v, bins, tmp):
        wid = lax.axis_index("s") * 2 + lax.axis_index("c")
        pltpu.sync_copy(a_h.at[pl.ds(wid * EW, EW)], a_v)
        pltpu.sync_copy(d_h.at[pl.ds(wid * EW, EW)], d_v)

        neg = jnp.full((16,), -1e30, _F32)

        def zb(r, c):
            for k in range(8):
                bins[r, pl.ds(k * 16, 16)] = neg
            return c

        lax.fori_loop(0, BINR, zb, 0)

        def edge(j, c):
            for u in range(5):
                o = j * 80 + u * 16
                d = d_v[pl.ds(o, 16)]
                v = a_v[pl.ds(o, 16)]
                _bin_update(bins, tmp, d, v, True)
            return c

        lax.fori_loop(0, EW // 80, edge, 0)
        pltpu.sync_copy(bins, out.at[wid])

    scratch = [
        pltpu.VMEM((EW,), _F32),
        pltpu.VMEM((EW,), _I32),
        pltpu.VMEM((BINR, 128), _F32),
        pltpu.VMEM((BINR, 128), _I32),
    ]
    out_t = jax.ShapeDtypeStruct((NW, BINR, 128), _F32)
    return pl.kernel(body, out_type=out_t, mesh=_mesh(),
                     compiler_params=_SC_PARAMS,
                     scratch_types=scratch)(a, dst)


def _sc_expw1(a, dst, mx):
    def body(a_h, d_h, mx_h, w_o, wp_out, a_v, d_v, mxv, bins):
        wid = lax.axis_index("s") * 2 + lax.axis_index("c")
        pltpu.sync_copy(a_h.at[pl.ds(wid * EW, EW)], a_v)
        pltpu.sync_copy(d_h.at[pl.ds(wid * EW, EW)], d_v)
        pltpu.sync_copy(mx_h, mxv)

        zero = jnp.zeros((16,), _F32)

        def zb(r, c):
            for k in range(8):
                bins[r, pl.ds(k * 16, 16)] = zero
            return c

        lax.fori_loop(0, BINR, zb, 0)

        def edge(j, c):
            for u in range(5):
                o = j * 80 + u * 16
                d = d_v[pl.ds(o, 16)]
                aa = a_v[pl.ds(o, 16)]
                ri = lax.shift_right_logical(d, 7)
                ci = lax.bitwise_and(d, 127)
                m = plsc.load_gather(mxv, [ri, ci])
                w = jnp.exp(aa - m)
                a_v[pl.ds(o, 16)] = w
                plsc.addupdate_scatter(bins, [ri, ci], w)
            return c

        lax.fori_loop(0, EW // 80, edge, 0)
        pltpu.sync_copy(a_v, w_o.at[pl.ds(wid * EW, EW)])
        pltpu.sync_copy(bins, wp_out.at[wid])

    scratch = [
        pltpu.VMEM((EW,), _F32),
        pltpu.VMEM((EW,), _I32),
        pltpu.VMEM((BINR, 128), _F32),
        pltpu.VMEM((BINR, 128), _F32),
    ]
    out_t = (jax.ShapeDtypeStruct((E,), _F32),
             jax.ShapeDtypeStruct((NW, BINR, 128), _F32))
    return pl.kernel(body, out_type=out_t, mesh=_mesh(),
                     compiler_params=_SC_PARAMS,
                     scratch_types=scratch)(a, dst, mx)


def _sc_scatter1(msg, idx):
    def body(m_h, i_h, out, *rest):
        rbuf = list(rest[0:4])
        ibuf = list(rest[4:8])
        rsem = list(rest[8:12])
        isem = list(rest[12:16])
        acc = rest[16]
        cid = lax.axis_index("c")
        sid = lax.axis_index("s")
        wid = sid * 2 + cid
        ebase = wid * EW

        zero = jnp.zeros((16,), _F32)

        def zrow(r, c):
            for k in range(H // 16):
                rbuf[0][r, pl.ds(k * 16, 16)] = zero
            return c

        lax.fori_loop(0, CH, zrow, 0)

        def zacc(q, c):
            pltpu.sync_copy(rbuf[0], acc.at[pl.ds(sid * 640 + q * CH, CH)])
            return c

        lax.fori_loop(0, 8, zacc, 0)
        plsc.subcore_barrier()

        def chunk(j):
            return m_h.at[pl.ds(ebase + j * CH, CH)]

        def ichunk(j):
            return i_h.at[pl.ds(ebase + j * CH, CH)]

        def start(b, j):
            pltpu.async_copy(chunk(j), rbuf[b], rsem[b])
            pltpu.async_copy(ichunk(j), ibuf[b], isem[b])

        def do_chunk(b, j):
            pltpu.make_async_copy(chunk(j), rbuf[b], rsem[b]).wait()
            pltpu.make_async_copy(ichunk(j), ibuf[b], isem[b]).wait()
            pltpu.sync_copy(rbuf[b], acc.at[ibuf[b]], add=True)

            @pl.when(j + 4 < NCH)
            def _():
                start(b, j + 4)

        for b in range(4):
            start(b, b)

        def grp(g, c):
            for b in range(4):
                do_chunk(b, 4 * g + b)
            return c

        lax.fori_loop(0, (NCH - 1) // 4, grp, 0)
        do_chunk(0, NCH - 1)
        plsc.subcore_barrier()

        def dump(q, c):
            r0 = sid * 640 + q * CH
            pltpu.sync_copy(acc.at[pl.ds(r0, CH)], rbuf[0])
            pltpu.sync_copy(rbuf[0], out.at[cid, pl.ds(r0, CH)])
            return c

        lax.fori_loop(0, 8, dump, 0)

    scratch = (
        [pltpu.VMEM((CH, H), _F32)] * 4
        + [pltpu.VMEM((CH,), _I32)] * 4
        + [pltpu.SemaphoreType.DMA] * 8
        + [pltpu.VMEM_SHARED((NPAD, H), _F32)]
    )
    out_t = jax.ShapeDtypeStruct((2, NPAD, H), _F32)
    return pl.kernel(body, out_type=out_t, mesh=_mesh(),
                     compiler_params=_SC_PARAMS,
                     scratch_types=scratch)(msg, idx)


def _sc_scatter3(msgs, idx):
    def body(m0, m1, m2, i_h, o0, o1, o2, *rest):
        rbuf = list(rest[0:4])
        ibuf = list(rest[4:8])
        rsem = list(rest[8:12])
        isem = list(rest[12:16])
        acc = rest[16]
        cid = lax.axis_index("c")
        sid = lax.axis_index("s")
        wid = sid * 2 + cid
        ebase = wid * EW

        zero = jnp.zeros((16,), _F32)

        for m_h, out in ((m0, o0), (m1, o1), (m2, o2)):
            def zrow(r, c):
                for k in range(H // 16):
                    rbuf[0][r, pl.ds(k * 16, 16)] = zero
                return c

            lax.fori_loop(0, CH, zrow, 0)

            def zacc(q, c):
                pltpu.sync_copy(rbuf[0], acc.at[pl.ds(sid * 640 + q * CH,
                                                      CH)])
                return c

            lax.fori_loop(0, 8, zacc, 0)
            plsc.subcore_barrier()

            def chunk(j):
                return m_h.at[pl.ds(ebase + j * CH, CH)]

            def ichunk(j):
                return i_h.at[pl.ds(ebase + j * CH, CH)]

            def start(b, j):
                pltpu.async_copy(chunk(j), rbuf[b], rsem[b])
                pltpu.async_copy(ichunk(j), ibuf[b], isem[b])

            def do_chunk(b, j):
                pltpu.make_async_copy(chunk(j), rbuf[b], rsem[b]).wait()
                pltpu.make_async_copy(ichunk(j), ibuf[b], isem[b]).wait()
                pltpu.sync_copy(rbuf[b], acc.at[ibuf[b]], add=True)

                @pl.when(j + 4 < NCH)
                def _():
                    start(b, j + 4)

            for b in range(4):
                start(b, b)

            def grp(g, c):
                for b in range(4):
                    do_chunk(b, 4 * g + b)
                return c

            lax.fori_loop(0, (NCH - 1) // 4, grp, 0)
            do_chunk(0, NCH - 1)
            plsc.subcore_barrier()

            def dump(q, c):
                r0 = sid * 640 + q * CH
                pltpu.sync_copy(acc.at[pl.ds(r0, CH)], rbuf[0])
                pltpu.sync_copy(rbuf[0], out.at[cid, pl.ds(r0, CH)])
                return c

            lax.fori_loop(0, 8, dump, 0)
            plsc.subcore_barrier()

    scratch = (
        [pltpu.VMEM((CH, H), _F32)] * 4
        + [pltpu.VMEM((CH,), _I32)] * 4
        + [pltpu.SemaphoreType.DMA] * 8
        + [pltpu.VMEM_SHARED((NPAD, H), _F32)]
    )
    out_t = tuple(jax.ShapeDtypeStruct((2, NPAD, H), _F32) for _ in range(3))
    return pl.kernel(body, out_type=out_t, mesh=_mesh(), compiler_params=_SC_PARAMS,
                     scratch_types=scratch)(msgs[0], msgs[1], msgs[2], idx)


_TC_PARAMS = pltpu.CompilerParams(
    dimension_semantics=("arbitrary", "arbitrary"))


def _gelu(x):
    return 0.5 * x * (1.0 + lax.erf(x * 0.7071067811865476))


def _mlp(xin, w1, b1, w2, b2):
    y = _gelu(jnp.dot(xin, w1, preferred_element_type=_F32) + b1)
    return jnp.dot(y, w2, preferred_element_type=_F32) + b2


def _full_spec(shape):
    nd = len(shape)
    return pl.BlockSpec(shape, lambda p, i: (0,) * nd)


def _bn_phases(p, i, y, rows, stats, out_ref, post=None):
    @pl.when(jnp.logical_and(p == 0, i == 0))
    def _():
        stats[...] = jnp.zeros_like(stats)

    @pl.when(p == 0)
    def _():
        stats[0:1, :] += jnp.sum(y, axis=0, keepdims=True)
        stats[1:2, :] += jnp.sum(y * y, axis=0, keepdims=True)
        out_ref[...] = y

    @pl.when(p == 1)
    def _():
        m = stats[0:1, :] / rows
        v = stats[1:2, :] / rows - m * m
        yn = (y - m) * lax.rsqrt(v + 1e-5)
        out_ref[...] = yn
        if post is not None:
            post(yn)


def _tc_red(part, is_max):
    def body(p_ref, o_ref):
        x = p_ref[...]
        o_ref[...] = jnp.max(x, axis=0) if is_max else jnp.sum(x, axis=0)

    return pl.pallas_call(
        body,
        grid=(1,),
        in_specs=[pl.BlockSpec((NW, BINR, 128), lambda i: (0, 0, 0))],
        out_specs=pl.BlockSpec((BINR, 128), lambda i: (0, 0)),
        out_shape=jax.ShapeDtypeStruct((BINR, 128), _F32),
    )(part)


def _tc_enc(xp, w1, b1, w2, b2, rows, br):
    cin = xp.shape[1]
    nb = rows // br

    def body(x_ref, w1r, b1r, w2r, b2r, out_ref, stats):
        p = pl.program_id(0)
        i = pl.program_id(1)
        y = _mlp(x_ref[...], w1r[...], b1r[...], w2r[...], b2r[...])
        _bn_phases(p, i, y, float(rows), stats, out_ref)

    return pl.pallas_call(
        body,
        grid=(2, nb),
        in_specs=[
            pl.BlockSpec((br, cin), lambda p, i: (i, 0)),
            _full_spec((cin, H)),
            _full_spec((H,)),
            _full_spec((H, H)),
            _full_spec((H,)),
        ],
        out_specs=pl.BlockSpec((br, H), lambda p, i: (i, 0)),
        out_shape=jax.ShapeDtypeStruct((rows, H), _F32),
        scratch_shapes=[pltpu.VMEM((8, H), _F32)],
        compiler_params=_TC_PARAMS,
    )(xp, w1, b1, w2, b2)


def _tc_logits(gs, gd, eh):
    nb = E // BRE

    def body(s_ref, d_ref, e_ref, o_ref):
        o_ref[...] = jnp.sum((s_ref[...] + e_ref[...]) * d_ref[...],
                             axis=1, keepdims=True)

    spec = pl.BlockSpec((BRE, H), lambda i: (i, 0))
    return pl.pallas_call(
        body,
        grid=(nb,),
        in_specs=[spec, spec, spec],
        out_specs=pl.BlockSpec((BRE, 1), lambda i: (i, 0)),
        out_shape=jax.ShapeDtypeStruct((E, 1), _F32),
    )(gs, gd, eh)


def _tc_msgw(gs, w):
    nb = E // BRE

    def body(s_ref, w_ref, o_ref):
        o_ref[...] = s_ref[...] * w_ref[...]

    return pl.pallas_call(
        body,
        grid=(nb,),
        in_specs=[
            pl.BlockSpec((BRE, H), lambda i: (i, 0)),
            pl.BlockSpec((BRE, 1), lambda i: (i, 0)),
        ],
        out_specs=pl.BlockSpec((BRE, H), lambda i: (i, 0)),
        out_shape=jax.ShapeDtypeStruct((E, H), _F32),
    )(gs, w)


def _tc_node_update(rows_p, wsum, nh, eps, w1, b1, w2, b2, bid2, want_gh):
    nb = N // BRN

    def body(rp_ref, ws_ref, nh_ref, eps_ref, w1r, b1r, w2r, b2r, bid_ref,
             out_ref, nz_ref, *rest):
        p = pl.program_id(0)
        i = pl.program_id(1)
        rp = rp_ref[...]
        denom = ws_ref[...] + 1e-16
        nz = (rp[0] + rp[1]) / denom
        nz_ref[...] = nz
        xin = (1.0 + eps_ref[...]) * nh_ref[...] + nz
        y = _mlp(xin, w1r[...], b1r[...], w2r[...], b2r[...])
        if want_gh:
            gh_ref, stats, acc = rest

            @pl.when(jnp.logical_and(p == 1, i == 0))
            def _():
                acc[...] = jnp.zeros_like(acc)

            def post(yn):
                oh = (bid_ref[...] ==
                      lax.broadcasted_iota(_I32, (1, G), 1)).astype(_F32)
                acc[...] += lax.dot_general(
                    oh, yn, (((0,), (0,)), ((), ())),
                    preferred_element_type=_F32)
                gh_ref[...] = acc[...]
        else:
            (stats,) = rest
            post = None
        _bn_phases(p, i, y, float(N), stats, out_ref, post=post)

    out_shape = [jax.ShapeDtypeStruct((N, H), _F32),
                 jax.ShapeDtypeStruct((N, H), _F32)]
    out_specs = [pl.BlockSpec((BRN, H), lambda p, i: (i, 0)),
                 pl.BlockSpec((BRN, H), lambda p, i: (i, 0))]
    scratch = [pltpu.VMEM((8, H), _F32)]
    if want_gh:
        out_shape.append(jax.ShapeDtypeStruct((G, H), _F32))
        out_specs.append(pl.BlockSpec((G, H), lambda p, i: (0, 0)))
        scratch.append(pltpu.VMEM((G, H), _F32))

    return pl.pallas_call(
        body,
        grid=(2, nb),
        in_specs=[
            pl.BlockSpec((2, BRN, H), lambda p, i: (0, i, 0)),
            pl.BlockSpec((BRN, 1), lambda p, i: (i, 0)),
            pl.BlockSpec((BRN, H), lambda p, i: (i, 0)),
            _full_spec((H,)),
            _full_spec((H, H)),
            _full_spec((H,)),
            _full_spec((H, H)),
            _full_spec((H,)),
            pl.BlockSpec((BRN, 1), lambda p, i: (i, 0)),
        ],
        out_specs=out_specs,
        out_shape=out_shape,
        scratch_shapes=scratch,
        compiler_params=_TC_PARAMS,
    )(rows_p, wsum, nh, eps, w1, b1, w2, b2, bid2)


def _tc_edge_update(eh, zs, zd, eps, w1, b1, w2, b2):
    nb = E // BRE

    def body(eh_ref, zs_ref, zd_ref, eps_ref, w1r, b1r, w2r, b2r, out_ref,
             stats):
        p = pl.program_id(0)
        i = pl.program_id(1)
        xin = ((1.0 + eps_ref[...]) * eh_ref[...] + zs_ref[...] - zd_ref[...])
        y = _mlp(xin, w1r[...], b1r[...], w2r[...], b2r[...])
        _bn_phases(p, i, y, float(E), stats, out_ref)

    spec = pl.BlockSpec((BRE, H), lambda p, i: (i, 0))
    return pl.pallas_call(
        body,
        grid=(2, nb),
        in_specs=[spec, spec, spec, _full_spec((H,)), _full_spec((H, H)),
                  _full_spec((H,)), _full_spec((H, H)), _full_spec((H,))],
        out_specs=spec,
        out_shape=jax.ShapeDtypeStruct((E, H), _F32),
        scratch_shapes=[pltpu.VMEM((8, H), _F32)],
        compiler_params=_TC_PARAMS,
    )(eh, zs, zd, eps, w1, b1, w2, b2)


_TAGS = ("n", "e", "g")


def kernel(x, edge_attr, weights, edge_index, batch_ids):
    src = edge_index[0]
    dst = edge_index[1]
    bid2 = batch_ids.reshape(N, 1)
    xp = jnp.pad(x, ((0, 0), (0, 16 - x.shape[1])))
    ep = jnp.pad(edge_attr, ((0, 0), (0, 8 - edge_attr.shape[1])))

    def enc(arr, layers, rows, br, cin):
        w1 = jnp.pad(layers[0]["W"], ((0, cin - layers[0]["W"].shape[0]),
                                      (0, 0)))
        return _tc_enc(arr, w1, layers[0]["b"], layers[1]["W"],
                       layers[1]["b"], rows, br)

    nh = {}
    eh = {}
    for tag in _TAGS:
        nh[tag] = enc(xp, weights["nf_lin_for_" + tag], N, BRN, 16)
        eh[tag] = enc(ep, weights["ef_lin_for_" + tag], E, BRE, 8)

    final = {}
    for li in range(DEPTH):
        last = li == DEPTH - 1
        nz = {}
        for tag in _TAGS:
            p = weights["gnn_" + tag]
            nl = p["nf_lin"]
            g_s, g_d = _sc_gather_multi([nh[tag]], src, dst)
            a1 = _tc_logits(g_s, g_d, eh[tag]).reshape(E)
            mx = _tc_red(_sc_segmax1(a1, dst), True)
            w, wp = _sc_expw1(a1, dst, mx)
            wsum = _tc_red(wp, False).reshape(NPAD, 1)
            msg = _tc_msgw(g_s, w.reshape(E, 1))
            rows_p = _sc_scatter1(msg, dst)
            want_gh = tag == "g" and last
            res = _tc_node_update(rows_p, wsum, nh[tag], p["nf_eps"],
                                  nl[0]["W"], nl[0]["b"], nl[1]["W"],
                                  nl[1]["b"], bid2, want_gh)
            nh[tag], nz[tag] = res[0], res[1]
            if want_gh:
                final["g"] = res[2]
        upd = ("e",) if last else _TAGS
        for tag in upd:
            p = weights["gnn_" + tag]
            el = p["ef_lin"]
            zs, zd = _sc_gather_multi([nz[tag]], src, dst)
            eh[tag] = _tc_edge_update(eh[tag], zs, zd, p["ef_eps"],
                                      el[0]["W"], el[0]["b"],
                                      el[1]["W"], el[1]["b"])
    final["n"] = nh["n"]
    final["e"] = eh["e"]
    return (final["n"], final["e"], final["g"])


# speedup vs baseline: 1.0815x; 1.0001x over previous
import jax
import jax.numpy as jnp
from jax import lax
from jax.experimental import pallas as pl
from jax.experimental.pallas import tpu as pltpu
from jax.experimental.pallas import tpu_sc as plsc

N = 10000
E = 320000
H = 128
G = 64
DEPTH = 3

NW = 32
EW = E // NW
CH = 80
NCH = EW // CH
NG = NCH // 5
BINR = 80
NPAD = 10240

BRE = 1280
BRN = 2000

_F32 = jnp.float32
_I32 = jnp.int32


def _mesh():
    return plsc.VectorSubcoreMesh(core_axis_name="c", subcore_axis_name="s")


_SC_PARAMS = pltpu.CompilerParams(needs_layout_passes=False)


def _sc_gather_multi(tables, idxa, idxb):
    nt = len(tables)

    def body(*refs):
        tabs = refs[:nt]
        ia, ib = refs[nt], refs[nt + 1]
        outs = refs[nt + 2:nt + 2 + 2 * nt]
        rest = refs[nt + 2 + 2 * nt:]
        iva, ivb = rest[0], rest[1]
        rest = rest[2:]
        bufs = [list(rest[0:5]), list(rest[5:10])]
        gsem = [list(rest[10:15]), list(rest[15:20])]
        osem = [list(rest[20:25]), list(rest[25:30])]
        wid = lax.axis_index("s") * 2 + lax.axis_index("c")
        ebase = wid * EW
        pltpu.sync_copy(ia.at[pl.ds(ebase, EW)], iva)
        pltpu.sync_copy(ib.at[pl.ds(ebase, EW)], ivb)

        def stream(tab, iv, out):
            def chunk(j):
                return out.at[pl.ds(ebase + j * CH, CH)]

            for b in range(5):
                pltpu.async_copy(tab.at[iv.at[pl.ds(b * CH, CH)]],
                                 bufs[0][b], gsem[0][b])

            def emit(bank, g):
                ob_ = 1 - bank
                for b in range(5):
                    @pl.when(g > 0)
                    def _():
                        pltpu.make_async_copy(
                            bufs[ob_][b], chunk(0), osem[ob_][b]).wait()

                    @pl.when(g < NG - 1)
                    def _():
                        pltpu.async_copy(
                            tab.at[iv.at[pl.ds((5 * (g + 1) + b) * CH, CH)]],
                            bufs[ob_][b], gsem[ob_][b])
                for b in range(5):
                    pltpu.make_async_copy(
                        tab.at[iv.at[pl.ds((5 * g + b) * CH, CH)]],
                        bufs[bank][b], gsem[bank][b]).wait()
                    pltpu.async_copy(bufs[bank][b], chunk(5 * g + b),
                                     osem[bank][b])

            def grp(g, c):
                @pl.when(g % 2 == 0)
                def _():
                    emit(0, g)

                @pl.when(g % 2 == 1)
                def _():
                    emit(1, g)
                return c

            lax.fori_loop(0, NG, grp, 0)
            bank = (NG - 1) % 2
            for b in range(5):
                pltpu.make_async_copy(bufs[bank][b], chunk(0),
                                      osem[bank][b]).wait()

        for t in range(nt):
            stream(tabs[t], iva, outs[2 * t])
            stream(tabs[t], ivb, outs[2 * t + 1])

    scratch = (
        [pltpu.VMEM((EW,), _I32)] * 2
        + [pltpu.VMEM((CH, H), _F32)] * 10
        + [pltpu.SemaphoreType.DMA] * 20
    )
    out_t = tuple(jax.ShapeDtypeStruct((E, H), _F32) for _ in range(2 * nt))
    return pl.kernel(body, out_type=out_t, mesh=_mesh(),
                     compiler_params=_SC_PARAMS,
                     scratch_types=scratch)(*tables, idxa, idxb)


def _bin_update(bins, tmp, d, vv, is_max):
    ri = lax.shift_right_logical(d, 7)
    ci = lax.bitwise_and(d, 127)
    iot = lax.iota(_I32, 16)
    ones = jnp.ones((16,), _I32)
    zeros = jnp.zeros((16,), _I32)

    def round_(rem):
        remb = rem != 0
        plsc.store_scatter(tmp, [ri, ci], iot, mask=remb)
        back = plsc.load_gather(tmp, [ri, ci])
        win = jnp.logical_and(remb, back == iot)
        cur = plsc.load_gather(bins, [ri, ci])
        upd = jnp.maximum(cur, vv) if is_max else cur + vv
        plsc.store_scatter(bins, [ri, ci], upd, mask=win)
        return jnp.where(jnp.logical_and(remb, jnp.logical_not(win)),
                         ones, zeros)

    rem0 = round_(ones)

    @pl.when(jnp.any(rem0 != 0))
    def _():
        def rounds(r, rm):
            return round_(rm)

        lax.fori_loop(0, 15, rounds, rem0)


def _sc_segmax1(a, dst):
    def body(a_h, d_h, out, a_v, d_v, bins, tmp):
        wid = lax.axis_index("s") * 2 + lax.axis_index("c")
        pltpu.sync_copy(a_h.at[pl.ds(wid * EW, EW)], a_v)
        pltpu.sync_copy(d_h.at[pl.ds(wid * EW, EW)], d_v)

        neg = jnp.full((16,), -1e30, _F32)

        def zb(r, c):
            for k in range(8):
                bins[r, pl.ds(k * 16, 16)] = neg
            return c

        lax.fori_loop(0, BINR, zb, 0)

        def edge(j, c):
            for u in range(5):
                o = j * 80 + u * 16
                d = d_v[pl.ds(o, 16)]
                v = a_v[pl.ds(o, 16)]
                _bin_update(bins, tmp, d, v, True)
            return c

        lax.fori_loop(0, EW // 80, edge, 0)
        pltpu.sync_copy(bins, out.at[wid])

    scratch = [
        pltpu.VMEM((EW,), _F32),
        pltpu.VMEM((EW,), _I32),
        pltpu.VMEM((BINR, 128), _F32),
        pltpu.VMEM((BINR, 128), _I32),
    ]
    out_t = jax.ShapeDtypeStruct((NW, BINR, 128), _F32)
    return pl.kernel(body, out_type=out_t, mesh=_mesh(),
                     compiler_params=_SC_PARAMS,
                     scratch_types=scratch)(a, dst)


def _sc_expw1(a, dst, mx):
    def body(a_h, d_h, mx_h, w_o, wp_out, a_v, d_v, mxv, bins):
        wid = lax.axis_index("s") * 2 + lax.axis_index("c")
        pltpu.sync_copy(a_h.at[pl.ds(wid * EW, EW)], a_v)
        pltpu.sync_copy(d_h.at[pl.ds(wid * EW, EW)], d_v)
        pltpu.sync_copy(mx_h, mxv)

        zero = jnp.zeros((16,), _F32)

        def zb(r, c):
            for k in range(8):
                bins[r, pl.ds(k * 16, 16)] = zero
            return c

        lax.fori_loop(0, BINR, zb, 0)

        def edge(j, c):
            for u in range(5):
                o = j * 80 + u * 16
                d = d_v[pl.ds(o, 16)]
                aa = a_v[pl.ds(o, 16)]
                ri = lax.shift_right_logical(d, 7)
                ci = lax.bitwise_and(d, 127)
                m = plsc.load_gather(mxv, [ri, ci])
                w = jnp.exp(aa - m)
                a_v[pl.ds(o, 16)] = w
                plsc.addupdate_scatter(bins, [ri, ci], w)
            return c

        lax.fori_loop(0, EW // 80, edge, 0)
        pltpu.sync_copy(a_v, w_o.at[pl.ds(wid * EW, EW)])
        pltpu.sync_copy(bins, wp_out.at[wid])

    scratch = [
        pltpu.VMEM((EW,), _F32),
        pltpu.VMEM((EW,), _I32),
        pltpu.VMEM((BINR, 128), _F32),
        pltpu.VMEM((BINR, 128), _F32),
    ]
    out_t = (jax.ShapeDtypeStruct((E,), _F32),
             jax.ShapeDtypeStruct((NW, BINR, 128), _F32))
    return pl.kernel(body, out_type=out_t, mesh=_mesh(),
                     compiler_params=_SC_PARAMS,
                     scratch_types=scratch)(a, dst, mx)


def _sc_scatter1(msg, idx):
    def body(m_h, i_h, out, *rest):
        rbuf = list(rest[0:4])
        ibuf = list(rest[4:8])
        rsem = list(rest[8:12])
        isem = list(rest[12:16])
        acc = rest[16]
        cid = lax.axis_index("c")
        sid = lax.axis_index("s")
        wid = sid * 2 + cid
        ebase = wid * EW

        zero = jnp.zeros((16,), _F32)

        def zrow(r, c):
            for k in range(H // 16):
                rbuf[0][r, pl.ds(k * 16, 16)] = zero
            return c

        lax.fori_loop(0, CH, zrow, 0)

        def zacc(q, c):
            pltpu.sync_copy(rbuf[0], acc.at[pl.ds(sid * 640 + q * CH, CH)])
            return c

        lax.fori_loop(0, 8, zacc, 0)
        plsc.subcore_barrier()

        def chunk(j):
            return m_h.at[pl.ds(ebase + j * CH, CH)]

        def ichunk(j):
            return i_h.at[pl.ds(ebase + j * CH, CH)]

        def start(b, j):
            pltpu.async_copy(chunk(j), rbuf[b], rsem[b])
            pltpu.async_copy(ichunk(j), ibuf[b], isem[b])

        def do_chunk(b, j):
            pltpu.make_async_copy(chunk(j), rbuf[b], rsem[b]).wait()
            pltpu.make_async_copy(ichunk(j), ibuf[b], isem[b]).wait()
            pltpu.sync_copy(rbuf[b], acc.at[ibuf[b]], add=True)

            @pl.when(j + 4 < NCH)
            def _():
                start(b, j + 4)

        for b in range(4):
            start(b, b)

        def grp(g, c):
            for b in range(4):
                do_chunk(b, 4 * g + b)
            return c

        lax.fori_loop(0, (NCH - 1) // 4, grp, 0)
        do_chunk(0, NCH - 1)
        plsc.subcore_barrier()

        def dump(q, c):
            r0 = sid * 640 + q * CH
            pltpu.sync_copy(acc.at[pl.ds(r0, CH)], rbuf[0])
            pltpu.sync_copy(rbuf[0], out.at[cid, pl.ds(r0, CH)])
            return c

        lax.fori_loop(0, 8, dump, 0)

    scratch = (
        [pltpu.VMEM((CH, H), _F32)] * 4
        + [pltpu.VMEM((CH,), _I32)] * 4
        + [pltpu.SemaphoreType.DMA] * 8
        + [pltpu.VMEM_SHARED((NPAD, H), _F32)]
    )
    out_t = jax.ShapeDtypeStruct((2, NPAD, H), _F32)
    return pl.kernel(body, out_type=out_t, mesh=_mesh(),
                     compiler_params=_SC_PARAMS,
                     scratch_types=scratch)(msg, idx)


_TC_PARAMS = pltpu.CompilerParams(
    dimension_semantics=("arbitrary", "arbitrary"))


def _gelu(x):
    return 0.5 * x * (1.0 + lax.erf(x * 0.7071067811865476))


def _mlp(xin, w1, b1, w2, b2):
    y = _gelu(jnp.dot(xin, w1, preferred_element_type=_F32) + b1)
    return jnp.dot(y, w2, preferred_element_type=_F32) + b2


def _full_spec(shape):
    nd = len(shape)
    return pl.BlockSpec(shape, lambda p, i: (0,) * nd)


def _bn_phases(p, i, y, rows, stats, out_ref, post=None):
    @pl.when(jnp.logical_and(p == 0, i == 0))
    def _():
        stats[...] = jnp.zeros_like(stats)

    @pl.when(p == 0)
    def _():
        stats[0:1, :] += jnp.sum(y, axis=0, keepdims=True)
        stats[1:2, :] += jnp.sum(y * y, axis=0, keepdims=True)
        out_ref[...] = y

    @pl.when(p == 1)
    def _():
        m = stats[0:1, :] / rows
        v = stats[1:2, :] / rows - m * m
        yn = (y - m) * lax.rsqrt(v + 1e-5)
        out_ref[...] = yn
        if post is not None:
            post(yn)


def _tc_red(part, is_max):
    def body(p_ref, o_ref):
        x = p_ref[...]
        o_ref[...] = jnp.max(x, axis=0) if is_max else jnp.sum(x, axis=0)

    return pl.pallas_call(
        body,
        grid=(1,),
        in_specs=[pl.BlockSpec((NW, BINR, 128), lambda i: (0, 0, 0))],
        out_specs=pl.BlockSpec((BINR, 128), lambda i: (0, 0)),
        out_shape=jax.ShapeDtypeStruct((BINR, 128), _F32),
    )(part)


def _tc_enc(xp, w1, b1, w2, b2, rows, br):
    cin = xp.shape[1]
    nb = rows // br

    def body(x_ref, w1r, b1r, w2r, b2r, out_ref, stats):
        p = pl.program_id(0)
        i = pl.program_id(1)
        y = _mlp(x_ref[...], w1r[...], b1r[...], w2r[...], b2r[...])
        _bn_phases(p, i, y, float(rows), stats, out_ref)

    return pl.pallas_call(
        body,
        grid=(2, nb),
        in_specs=[
            pl.BlockSpec((br, cin), lambda p, i: (i, 0)),
            _full_spec((cin, H)),
            _full_spec((H,)),
            _full_spec((H, H)),
            _full_spec((H,)),
        ],
        out_specs=pl.BlockSpec((br, H), lambda p, i: (i, 0)),
        out_shape=jax.ShapeDtypeStruct((rows, H), _F32),
        scratch_shapes=[pltpu.VMEM((8, H), _F32)],
        compiler_params=_TC_PARAMS,
    )(xp, w1, b1, w2, b2)


def _tc_logits(gs, gd, eh):
    nb = E // BRE

    def body(s_ref, d_ref, e_ref, o_ref):
        o_ref[...] = jnp.sum((s_ref[...] + e_ref[...]) * d_ref[...],
                             axis=1, keepdims=True)

    spec = pl.BlockSpec((BRE, H), lambda i: (i, 0))
    return pl.pallas_call(
        body,
        grid=(nb,),
        in_specs=[spec, spec, spec],
        out_specs=pl.BlockSpec((BRE, 1), lambda i: (i, 0)),
        out_shape=jax.ShapeDtypeStruct((E, 1), _F32),
    )(gs, gd, eh)


def _tc_msgw(gs, w):
    nb = E // BRE

    def body(s_ref, w_ref, o_ref):
        o_ref[...] = s_ref[...] * w_ref[...]

    return pl.pallas_call(
        body,
        grid=(nb,),
        in_specs=[
            pl.BlockSpec((BRE, H), lambda i: (i, 0)),
            pl.BlockSpec((BRE, 1), lambda i: (i, 0)),
        ],
        out_specs=pl.BlockSpec((BRE, H), lambda i: (i, 0)),
        out_shape=jax.ShapeDtypeStruct((E, H), _F32),
    )(gs, w)


def _tc_node_update(rows_p, wsum, nh, eps, w1, b1, w2, b2, bid2, want_gh):
    nb = N // BRN

    def body(rp_ref, ws_ref, nh_ref, eps_ref, w1r, b1r, w2r, b2r, bid_ref,
             out_ref, nz_ref, *rest):
        p = pl.program_id(0)
        i = pl.program_id(1)
        rp = rp_ref[...]
        denom = ws_ref[...] + 1e-16
        nz = (rp[0] + rp[1]) / denom
        nz_ref[...] = nz
        xin = (1.0 + eps_ref[...]) * nh_ref[...] + nz
        y = _mlp(xin, w1r[...], b1r[...], w2r[...], b2r[...])
        if want_gh:
            gh_ref, stats, acc = rest

            @pl.when(jnp.logical_and(p == 1, i == 0))
            def _():
                acc[...] = jnp.zeros_like(acc)

            def post(yn):
                oh = (bid_ref[...] ==
                      lax.broadcasted_iota(_I32, (1, G), 1)).astype(_F32)
                acc[...] += lax.dot_general(
                    oh, yn, (((0,), (0,)), ((), ())),
                    preferred_element_type=_F32)
                gh_ref[...] = acc[...]
        else:
            (stats,) = rest
            post = None
        _bn_phases(p, i, y, float(N), stats, out_ref, post=post)

    out_shape = [jax.ShapeDtypeStruct((N, H), _F32),
                 jax.ShapeDtypeStruct((N, H), _F32)]
    out_specs = [pl.BlockSpec((BRN, H), lambda p, i: (i, 0)),
                 pl.BlockSpec((BRN, H), lambda p, i: (i, 0))]
    scratch = [pltpu.VMEM((8, H), _F32)]
    if want_gh:
        out_shape.append(jax.ShapeDtypeStruct((G, H), _F32))
        out_specs.append(pl.BlockSpec((G, H), lambda p, i: (0, 0)))
        scratch.append(pltpu.VMEM((G, H), _F32))

    return pl.pallas_call(
        body,
        grid=(2, nb),
        in_specs=[
            pl.BlockSpec((2, BRN, H), lambda p, i: (0, i, 0)),
            pl.BlockSpec((BRN, 1), lambda p, i: (i, 0)),
            pl.BlockSpec((BRN, H), lambda p, i: (i, 0)),
            _full_spec((H,)),
            _full_spec((H, H)),
            _full_spec((H,)),
            _full_spec((H, H)),
            _full_spec((H,)),
            pl.BlockSpec((BRN, 1), lambda p, i: (i, 0)),
        ],
        out_specs=out_specs,
        out_shape=out_shape,
        scratch_shapes=scratch,
        compiler_params=_TC_PARAMS,
    )(rows_p, wsum, nh, eps, w1, b1, w2, b2, bid2)


def _tc_edge_update(eh, zs, zd, eps, w1, b1, w2, b2):
    nb = E // BRE

    def body(eh_ref, zs_ref, zd_ref, eps_ref, w1r, b1r, w2r, b2r, out_ref,
             stats):
        p = pl.program_id(0)
        i = pl.program_id(1)
        xin = ((1.0 + eps_ref[...]) * eh_ref[...] + zs_ref[...] - zd_ref[...])
        y = _mlp(xin, w1r[...], b1r[...], w2r[...], b2r[...])
        _bn_phases(p, i, y, float(E), stats, out_ref)

    spec = pl.BlockSpec((BRE, H), lambda p, i: (i, 0))
    return pl.pallas_call(
        body,
        grid=(2, nb),
        in_specs=[spec, spec, spec, _full_spec((H,)), _full_spec((H, H)),
                  _full_spec((H,)), _full_spec((H, H)), _full_spec((H,))],
        out_specs=spec,
        out_shape=jax.ShapeDtypeStruct((E, H), _F32),
        scratch_shapes=[pltpu.VMEM((8, H), _F32)],
        compiler_params=_TC_PARAMS,
    )(eh, zs, zd, eps, w1, b1, w2, b2)


_TAGS = ("n", "e", "g")


def kernel(x, edge_attr, weights, edge_index, batch_ids):
    src = edge_index[0]
    dst = edge_index[1]
    bid2 = batch_ids.reshape(N, 1)
    xp = jnp.pad(x, ((0, 0), (0, 16 - x.shape[1])))
    ep = jnp.pad(edge_attr, ((0, 0), (0, 8 - edge_attr.shape[1])))

    def enc(arr, layers, rows, br, cin):
        w1 = jnp.pad(layers[0]["W"], ((0, cin - layers[0]["W"].shape[0]),
                                      (0, 0)))
        return _tc_enc(arr, w1, layers[0]["b"], layers[1]["W"],
                       layers[1]["b"], rows, br)

    nh = {}
    eh = {}
    for tag in _TAGS:
        nh[tag] = enc(xp, weights["nf_lin_for_" + tag], N, BRN, 16)
        eh[tag] = enc(ep, weights["ef_lin_for_" + tag], E, BRE, 8)

    final = {}
    for li in range(DEPTH):
        last = li == DEPTH - 1
        nz = {}
        for tag in _TAGS:
            p = weights["gnn_" + tag]
            nl = p["nf_lin"]
            g_s, g_d = _sc_gather_multi([nh[tag]], src, dst)
            a1 = _tc_logits(g_s, g_d, eh[tag]).reshape(E)
            mx = _tc_red(_sc_segmax1(a1, dst), True)
            w, wp = _sc_expw1(a1, dst, mx)
            wsum = _tc_red(wp, False).reshape(NPAD, 1)
            msg = _tc_msgw(g_s, w.reshape(E, 1))
            rows_p = _sc_scatter1(msg, dst)
            want_gh = tag == "g" and last
            res = _tc_node_update(rows_p, wsum, nh[tag], p["nf_eps"],
                                  nl[0]["W"], nl[0]["b"], nl[1]["W"],
                                  nl[1]["b"], bid2, want_gh)
            nh[tag], nz[tag] = res[0], res[1]
            if want_gh:
                final["g"] = res[2]
        upd = ("e",) if last else _TAGS
        for tag in upd:
            p = weights["gnn_" + tag]
            el = p["ef_lin"]
            zs, zd = _sc_gather_multi([nz[tag]], src, dst)
            eh[tag] = _tc_edge_update(eh[tag], zs, zd, p["ef_eps"],
                                      el[0]["W"], el[0]["b"],
                                      el[1]["W"], el[1]["b"])
    final["n"] = nh["n"]
    final["e"] = eh["e"]
    return (final["n"], final["e"], final["g"])
